# Initial kernel scaffold; baseline (speedup 1.0000x reference)
import jax
import jax.numpy as jnp
from jax.experimental import pallas as pl


def kernel(sc_x, fc_x, sc_edge_index, fc_edge_index, batch, sc1_Wrel, sc1_brel, sc1_Wroot, sc2_Wrel, sc2_brel, sc2_Wroot, fc1_Wrel, fc1_brel, fc1_Wroot, fc2_Wrel, fc2_brel, fc2_Wroot, W1, b1, W2, b2, W3, b3):
    raise NotImplementedError("write your pallas kernel here")


# speedup vs baseline: 3.2749x; 3.2749x over previous
import functools
import jax
import jax.numpy as jnp
from jax import lax
from jax.experimental import pallas as pl
from jax.experimental.pallas import tpu as pltpu
from jax.experimental.pallas import tpu_sc as plsc

N = 10000
E = 320000
F = 128
G = 64

_NC = 2
_NS = 16
_K = 80
_EP = E // _NS
_NCHUNK = _EP // _K
_RT = 624
_CZ = 208
_NCOPY = _RT // _CZ
_TAIL = N - _NS * _RT


def _sc_segsum_body(x_sc, x_fc, src_sc, dst_sc, src_fc, dst_fc,
                    out_sum,
                    idx_v, dst_v, rows_v, zero_v,
                    acc_sh, sem):
  c = lax.axis_index("c")
  s = lax.axis_index("s")

  def _init_rows(i, _):
    def _init_lane(j, _):
      zero_v[i, pl.ds(j * 16, 16)] = jnp.zeros((16,), jnp.float32)
      return 0
    lax.fori_loop(0, F // 16, _init_lane, 0)
    return 0
  lax.fori_loop(0, _CZ, _init_rows, 0)

  for j in range(_NCOPY):
    r0 = s * _RT + j * _CZ
    pltpu.sync_copy(zero_v, acc_sh.at[pl.ds(r0, _CZ)])

  @pl.when(s == 0)
  def _():
    pltpu.sync_copy(zero_v.at[pl.ds(0, _TAIL)],
                    acc_sh.at[pl.ds(_NS * _RT, _TAIL)])

  plsc.subcore_barrier()

  def _process(x_ref, src_ref, dst_ref):
    def _chunk(i, _):
      off = s * _EP + i * _K
      pltpu.sync_copy(src_ref.at[pl.ds(off, _K)], idx_v)
      pltpu.sync_copy(dst_ref.at[pl.ds(off, _K)], dst_v)
      pltpu.async_copy(x_ref.at[idx_v], rows_v, sem).wait()
      pltpu.sync_copy(rows_v, acc_sh.at[dst_v], add=True)
      return 0
    lax.fori_loop(0, _NCHUNK, _chunk, 0)

  @pl.when(c == 0)
  def _():
    _process(x_sc, src_sc, dst_sc)

  @pl.when(c == 1)
  def _():
    _process(x_fc, src_fc, dst_fc)

  plsc.subcore_barrier()

  for j in range(_NCOPY):
    r0 = s * _RT + j * _CZ
    pltpu.sync_copy(acc_sh.at[pl.ds(r0, _CZ)], zero_v)
    pltpu.sync_copy(zero_v, out_sum.at[c, pl.ds(r0, _CZ)])

  @pl.when(s == 0)
  def _():
    r0 = _NS * _RT
    pltpu.sync_copy(acc_sh.at[pl.ds(r0, _TAIL)], zero_v.at[pl.ds(0, _TAIL)])
    pltpu.sync_copy(zero_v.at[pl.ds(0, _TAIL)], out_sum.at[c, pl.ds(r0, _TAIL)])


def _sc_segsum(x_sc, x_fc, src_sc, dst_sc, src_fc, dst_fc):
  mesh = plsc.VectorSubcoreMesh(core_axis_name="c", subcore_axis_name="s",
                                num_cores=_NC, num_subcores=_NS)
  f = pl.kernel(
      _sc_segsum_body,
      mesh=mesh,
      out_type=jax.ShapeDtypeStruct((_NC, N, F), jnp.float32),
      scratch_types=[
          pltpu.VMEM((_K,), jnp.int32),
          pltpu.VMEM((_K,), jnp.int32),
          pltpu.VMEM((_K, F), jnp.float32),
          pltpu.VMEM((_CZ, F), jnp.float32),
          pltpu.VMEM_SHARED((N, F), jnp.float32),
          pltpu.SemaphoreType.DMA,
      ],
  )
  return f(x_sc, x_fc, src_sc, dst_sc, src_fc, dst_fc)


def _sc_segcnt_body(dst_sc, dst_fc, out_cnt,
                    dst_v, ones_v, zcnt_v, cnt_sh):
  c = lax.axis_index("c")
  s = lax.axis_index("s")

  def _init_rows(i, _):
    def _init_lane(j, _):
      zcnt_v[i, pl.ds(j * 16, 16)] = jnp.zeros((16,), jnp.float32)
      return 0
    lax.fori_loop(0, F // 16, _init_lane, 0)
    return 0
  lax.fori_loop(0, _CZ, _init_rows, 0)

  def _init_ones(i, _):
    def _init_lane(j, _):
      ones_v[i, pl.ds(j * 16, 16)] = jnp.ones((16,), jnp.float32)
      return 0
    lax.fori_loop(0, F // 16, _init_lane, 0)
    return 0
  lax.fori_loop(0, _K, _init_ones, 0)

  for j in range(_NCOPY):
    r0 = s * _RT + j * _CZ
    pltpu.sync_copy(zcnt_v, cnt_sh.at[pl.ds(r0, _CZ)])

  @pl.when(s == 0)
  def _():
    pltpu.sync_copy(zcnt_v.at[pl.ds(0, _TAIL)],
                    cnt_sh.at[pl.ds(_NS * _RT, _TAIL)])

  plsc.subcore_barrier()

  def _process(dst_ref):
    def _chunk(i, _):
      off = s * _EP + i * _K
      pltpu.sync_copy(dst_ref.at[pl.ds(off, _K)], dst_v)
      pltpu.sync_copy(ones_v, cnt_sh.at[dst_v], add=True)
      return 0
    lax.fori_loop(0, _NCHUNK, _chunk, 0)

  @pl.when(c == 0)
  def _():
    _process(dst_sc)

  @pl.when(c == 1)
  def _():
    _process(dst_fc)

  plsc.subcore_barrier()

  for j in range(_NCOPY):
    r0 = s * _RT + j * _CZ
    pltpu.sync_copy(cnt_sh.at[pl.ds(r0, _CZ)], zcnt_v)
    pltpu.sync_copy(zcnt_v, out_cnt.at[c, pl.ds(r0, _CZ)])

  @pl.when(s == 0)
  def _():
    r0 = _NS * _RT
    pltpu.sync_copy(cnt_sh.at[pl.ds(r0, _TAIL)], zcnt_v.at[pl.ds(0, _TAIL)])
    pltpu.sync_copy(zcnt_v.at[pl.ds(0, _TAIL)], out_cnt.at[c, pl.ds(r0, _TAIL)])


def _sc_segcnt(dst_sc, dst_fc):
  mesh = plsc.VectorSubcoreMesh(core_axis_name="c", subcore_axis_name="s",
                                num_cores=_NC, num_subcores=_NS)
  f = pl.kernel(
      _sc_segcnt_body,
      mesh=mesh,
      out_type=jax.ShapeDtypeStruct((_NC, N, F), jnp.float32),
      scratch_types=[
          pltpu.VMEM((_K,), jnp.int32),
          pltpu.VMEM((_K, F), jnp.float32),
          pltpu.VMEM((_CZ, F), jnp.float32),
          pltpu.VMEM_SHARED((N, F), jnp.float32),
      ],
  )
  return f(dst_sc, dst_fc)


_R = 2000
_GRID = N // _R


def _tc_layer_body(batch_ref, x_sc_ref, x_fc_ref, sum_ref, cnt_ref,
                   wrel_sc_ref, wroot_sc_ref, brel_sc_ref,
                   wrel_fc_ref, wroot_fc_ref, brel_fc_ref,
                   h_sc_ref, h_fc_ref, p_sc_ref, p_fc_ref):
  step = pl.program_id(0)
  batch = batch_ref[0, 0, :]
  onehot = (jax.lax.broadcasted_iota(jnp.int32, (G, _R), 0)
            == batch[None, :]).astype(jnp.float32)

  for b, (x_ref, wrel_ref, wroot_ref, brel_ref, h_ref, p_ref) in enumerate([
      (x_sc_ref, wrel_sc_ref, wroot_sc_ref, brel_sc_ref, h_sc_ref, p_sc_ref),
      (x_fc_ref, wrel_fc_ref, wroot_fc_ref, brel_fc_ref, h_fc_ref, p_fc_ref),
  ]):
    cnt = jnp.maximum(cnt_ref[b, :, 0:1], 1.0)
    agg = sum_ref[b] / cnt
    x = x_ref[...]
    h = jnp.dot(agg, wrel_ref[...], preferred_element_type=jnp.float32)
    h = h + jnp.dot(x, wroot_ref[...], preferred_element_type=jnp.float32)
    h = jnp.maximum(h + brel_ref[...][None, :], 0.0)
    h_ref[...] = h
    part = jnp.dot(onehot, h, preferred_element_type=jnp.float32)

    @pl.when(step == 0)
    def _():
      p_ref[...] = part

    @pl.when(step != 0)
    def _():
      p_ref[...] += part


def _tc_layer(batch3, x_sc, x_fc, sums, cnts,
              wrel_sc, wroot_sc, brel_sc, wrel_fc, wroot_fc, brel_fc):
  full = lambda shape: pl.BlockSpec(shape, lambda i: tuple(0 for _ in shape))
  return pl.pallas_call(
      _tc_layer_body,
      grid=(_GRID,),
      in_specs=[
          pl.BlockSpec((1, 1, _R), lambda i: (i, 0, 0)),
          pl.BlockSpec((_R, F), lambda i: (i, 0)),
          pl.BlockSpec((_R, F), lambda i: (i, 0)),
          pl.BlockSpec((_NC, _R, F), lambda i: (0, i, 0)),
          pl.BlockSpec((_NC, _R, F), lambda i: (0, i, 0)),
          full((F, F)), full((F, F)), full((F,)),
          full((F, F)), full((F, F)), full((F,)),
      ],
      out_specs=[
          pl.BlockSpec((_R, F), lambda i: (i, 0)),
          pl.BlockSpec((_R, F), lambda i: (i, 0)),
          pl.BlockSpec((G, F), lambda i: (0, 0)),
          pl.BlockSpec((G, F), lambda i: (0, 0)),
      ],
      out_shape=[
          jax.ShapeDtypeStruct((N, F), jnp.float32),
          jax.ShapeDtypeStruct((N, F), jnp.float32),
          jax.ShapeDtypeStruct((G, F), jnp.float32),
          jax.ShapeDtypeStruct((G, F), jnp.float32),
      ],
  )(batch3, x_sc, x_fc, sums, cnts,
    wrel_sc, wroot_sc, brel_sc, wrel_fc, wroot_fc, brel_fc)


def _tc_head_body(p1s_ref, p2s_ref, p1f_ref, p2f_ref,
                  w1_ref, b1_ref, w2_ref, b2_ref, w3_ref, b3_ref, out_ref):
  xcat = jnp.concatenate(
      [p1s_ref[...], p2s_ref[...], p1f_ref[...], p2f_ref[...]], axis=1)
  x = jnp.maximum(
      jnp.dot(xcat, w1_ref[...], preferred_element_type=jnp.float32)
      + b1_ref[...][None, :], 0.0)
  x = jnp.maximum(
      jnp.dot(x, w2_ref[...], preferred_element_type=jnp.float32)
      + b2_ref[...][None, :], 0.0)
  x = (jnp.dot(x, w3_ref[...], preferred_element_type=jnp.float32)
       + b3_ref[...][None, :])
  m = jnp.max(x, axis=-1, keepdims=True)
  lse = m + jnp.log(jnp.sum(jnp.exp(x - m), axis=-1, keepdims=True))
  out_ref[...] = x - lse


def _tc_head(p1s, p2s, p1f, p2f, w1, b1, w2, b2, w3, b3):
  return pl.pallas_call(
      _tc_head_body,
      out_shape=jax.ShapeDtypeStruct((G, 2), jnp.float32),
  )(p1s, p2s, p1f, p2f, w1, b1, w2, b2, w3, b3)


@jax.jit
def kernel(sc_x, fc_x, sc_edge_index, fc_edge_index, batch,
           sc1_Wrel, sc1_brel, sc1_Wroot, sc2_Wrel, sc2_brel, sc2_Wroot,
           fc1_Wrel, fc1_brel, fc1_Wroot, fc2_Wrel, fc2_brel, fc2_Wroot,
           W1, b1, W2, b2, W3, b3):
  src_sc, dst_sc = sc_edge_index[0], sc_edge_index[1]
  src_fc, dst_fc = fc_edge_index[0], fc_edge_index[1]
  batch3 = batch.reshape(_GRID, 1, _R)

  cnts = _sc_segcnt(dst_sc, dst_fc)
  sums1 = _sc_segsum(sc_x, fc_x, src_sc, dst_sc, src_fc, dst_fc)
  h1_sc, h1_fc, p1_sc, p1_fc = _tc_layer(
      batch3, sc_x, fc_x, sums1, cnts,
      sc1_Wrel, sc1_Wroot, sc1_brel, fc1_Wrel, fc1_Wroot, fc1_brel)

  sums2 = _sc_segsum(h1_sc, h1_fc, src_sc, dst_sc, src_fc, dst_fc)
  h2_sc, h2_fc, p2_sc, p2_fc = _tc_layer(
      batch3, h1_sc, h1_fc, sums2, cnts,
      sc2_Wrel, sc2_Wroot, sc2_brel, fc2_Wrel, fc2_Wroot, fc2_brel)

  return _tc_head(p1_sc, p2_sc, p1_fc, p2_fc, W1, b1, W2, b2, W3, b3)


# speedup vs baseline: 6.0355x; 1.8430x over previous
import functools
import jax
import jax.numpy as jnp
from jax import lax
from jax.experimental import pallas as pl
from jax.experimental.pallas import tpu as pltpu
from jax.experimental.pallas import tpu_sc as plsc

N = 10000
E = 320000
F = 128
G = 64

_NC = 2
_NS = 16
_K = 80
_EP = E // _NS
_NCHUNK = _EP // _K
_RT = 624
_CZ = 208
_NCOPY = _RT // _CZ
_TAIL = N - _NS * _RT


def _sc_segsum_body(x_sc, x_fc, src_sc, dst_sc, src_fc, dst_fc,
                    out_sum,
                    idx0, idx1, dst0, dst1, rows0, rows1, zero_v,
                    acc_sh, gsem0, gsem1, ssem0, ssem1, isem0, isem1):
  c = lax.axis_index("c")
  s = lax.axis_index("s")

  def _init_rows(i, _):
    def _init_lane(j, _):
      zero_v[i, pl.ds(j * 16, 16)] = jnp.zeros((16,), jnp.float32)
      return 0
    lax.fori_loop(0, F // 16, _init_lane, 0)
    return 0
  lax.fori_loop(0, _CZ, _init_rows, 0)

  for j in range(_NCOPY):
    r0 = s * _RT + j * _CZ
    pltpu.sync_copy(zero_v, acc_sh.at[pl.ds(r0, _CZ)])

  @pl.when(s == 0)
  def _():
    pltpu.sync_copy(zero_v.at[pl.ds(0, _TAIL)],
                    acc_sh.at[pl.ds(_NS * _RT, _TAIL)])

  plsc.subcore_barrier()

  def _process(x_ref, src_ref, dst_ref):
    bufs = ((idx0, dst0, rows0, gsem0, ssem0, isem0),
            (idx1, dst1, rows1, gsem1, ssem1, isem1))

    def idx_start(chunk, b):
      iv, dv, _, _, _, isem = bufs[b]
      off = s * _EP + chunk * _K
      pltpu.async_copy(src_ref.at[pl.ds(off, _K)], iv, isem)
      pltpu.async_copy(dst_ref.at[pl.ds(off, _K)], dv, isem)

    def idx_wait(b):
      iv, dv, _, _, _, isem = bufs[b]
      pltpu.make_async_copy(src_ref.at[pl.ds(0, _K)], iv, isem).wait()
      pltpu.make_async_copy(dst_ref.at[pl.ds(0, _K)], dv, isem).wait()

    def g_start(b):
      iv, _, rv, gsem, _, _ = bufs[b]
      pltpu.async_copy(x_ref.at[iv], rv, gsem)

    def g_wait(b):
      iv, _, rv, gsem, _, _ = bufs[b]
      pltpu.make_async_copy(x_ref.at[iv], rv, gsem).wait()

    def s_start(b):
      _, dv, rv, _, ssem, _ = bufs[b]
      pltpu.async_copy(rv, acc_sh.at[dv], ssem, add=True)

    def s_wait(b):
      _, dv, rv, _, ssem, _ = bufs[b]
      pltpu.make_async_copy(rv, acc_sh.at[dv], ssem).wait()

    idx_start(0, 0)
    idx_wait(0)
    idx_start(1, 1)
    g_start(0)

    def body(j, _):
      g_wait(0)
      s_start(0)
      idx_wait(1)
      g_start(1)
      s_wait(0)

      @pl.when(2 * j + 2 < _NCHUNK)
      def _():
        idx_start(2 * j + 2, 0)

      g_wait(1)
      s_start(1)

      @pl.when(2 * j + 2 < _NCHUNK)
      def _():
        idx_wait(0)
        g_start(0)

      s_wait(1)

      @pl.when(2 * j + 3 < _NCHUNK)
      def _():
        idx_start(2 * j + 3, 1)

      return 0
    lax.fori_loop(0, _NCHUNK // 2, body, 0)

  @pl.when(c == 0)
  def _():
    _process(x_sc, src_sc, dst_sc)

  @pl.when(c == 1)
  def _():
    _process(x_fc, src_fc, dst_fc)

  plsc.subcore_barrier()

  for j in range(_NCOPY):
    r0 = s * _RT + j * _CZ
    pltpu.sync_copy(acc_sh.at[pl.ds(r0, _CZ)], zero_v)
    pltpu.sync_copy(zero_v, out_sum.at[c, pl.ds(r0, _CZ)])

  @pl.when(s == 0)
  def _():
    r0 = _NS * _RT
    pltpu.sync_copy(acc_sh.at[pl.ds(r0, _TAIL)], zero_v.at[pl.ds(0, _TAIL)])
    pltpu.sync_copy(zero_v.at[pl.ds(0, _TAIL)], out_sum.at[c, pl.ds(r0, _TAIL)])


def _sc_segsum(x_sc, x_fc, src_sc, dst_sc, src_fc, dst_fc):
  mesh = plsc.VectorSubcoreMesh(core_axis_name="c", subcore_axis_name="s",
                                num_cores=_NC, num_subcores=_NS)
  f = pl.kernel(
      _sc_segsum_body,
      mesh=mesh,
      out_type=jax.ShapeDtypeStruct((_NC, N, F), jnp.float32),
      scratch_types=[
          pltpu.VMEM((_K,), jnp.int32),
          pltpu.VMEM((_K,), jnp.int32),
          pltpu.VMEM((_K,), jnp.int32),
          pltpu.VMEM((_K,), jnp.int32),
          pltpu.VMEM((_K, F), jnp.float32),
          pltpu.VMEM((_K, F), jnp.float32),
          pltpu.VMEM((_CZ, F), jnp.float32),
          pltpu.VMEM_SHARED((N, F), jnp.float32),
          pltpu.SemaphoreType.DMA,
          pltpu.SemaphoreType.DMA,
          pltpu.SemaphoreType.DMA,
          pltpu.SemaphoreType.DMA,
          pltpu.SemaphoreType.DMA,
          pltpu.SemaphoreType.DMA,
      ],
  )
  return f(x_sc, x_fc, src_sc, dst_sc, src_fc, dst_fc)


def _sc_segcnt_body(dst_sc, dst_fc, out_cnt,
                    dst0, dst1, ones_v, zcnt_v, cnt_sh,
                    ssem0, ssem1, isem0, isem1):
  c = lax.axis_index("c")
  s = lax.axis_index("s")

  def _init_rows(i, _):
    def _init_lane(j, _):
      zcnt_v[i, pl.ds(j * 16, 16)] = jnp.zeros((16,), jnp.float32)
      return 0
    lax.fori_loop(0, F // 16, _init_lane, 0)
    return 0
  lax.fori_loop(0, _CZ, _init_rows, 0)

  def _init_ones(i, _):
    def _init_lane(j, _):
      ones_v[i, pl.ds(j * 16, 16)] = jnp.ones((16,), jnp.float32)
      return 0
    lax.fori_loop(0, F // 16, _init_lane, 0)
    return 0
  lax.fori_loop(0, _K, _init_ones, 0)

  for j in range(_NCOPY):
    r0 = s * _RT + j * _CZ
    pltpu.sync_copy(zcnt_v, cnt_sh.at[pl.ds(r0, _CZ)])

  @pl.when(s == 0)
  def _():
    pltpu.sync_copy(zcnt_v.at[pl.ds(0, _TAIL)],
                    cnt_sh.at[pl.ds(_NS * _RT, _TAIL)])

  plsc.subcore_barrier()

  def _process(dst_ref):
    bufs = ((dst0, ssem0, isem0), (dst1, ssem1, isem1))

    def idx_start(chunk, b):
      dv, _, isem = bufs[b]
      off = s * _EP + chunk * _K
      pltpu.async_copy(dst_ref.at[pl.ds(off, _K)], dv, isem)

    def idx_wait(b):
      dv, _, isem = bufs[b]
      pltpu.make_async_copy(dst_ref.at[pl.ds(0, _K)], dv, isem).wait()

    def s_start(b):
      dv, ssem, _ = bufs[b]
      pltpu.async_copy(ones_v, cnt_sh.at[dv], ssem, add=True)

    def s_wait(b):
      dv, ssem, _ = bufs[b]
      pltpu.make_async_copy(ones_v, cnt_sh.at[dv], ssem).wait()

    idx_start(0, 0)
    idx_start(1, 1)

    def body(j, _):
      idx_wait(0)
      s_start(0)
      idx_wait(1)
      s_start(1)
      s_wait(0)

      @pl.when(2 * j + 2 < _NCHUNK)
      def _():
        idx_start(2 * j + 2, 0)

      s_wait(1)

      @pl.when(2 * j + 3 < _NCHUNK)
      def _():
        idx_start(2 * j + 3, 1)

      return 0
    lax.fori_loop(0, _NCHUNK // 2, body, 0)

  @pl.when(c == 0)
  def _():
    _process(dst_sc)

  @pl.when(c == 1)
  def _():
    _process(dst_fc)

  plsc.subcore_barrier()

  for j in range(_NCOPY):
    r0 = s * _RT + j * _CZ
    pltpu.sync_copy(cnt_sh.at[pl.ds(r0, _CZ)], zcnt_v)
    pltpu.sync_copy(zcnt_v, out_cnt.at[c, pl.ds(r0, _CZ)])

  @pl.when(s == 0)
  def _():
    r0 = _NS * _RT
    pltpu.sync_copy(cnt_sh.at[pl.ds(r0, _TAIL)], zcnt_v.at[pl.ds(0, _TAIL)])
    pltpu.sync_copy(zcnt_v.at[pl.ds(0, _TAIL)], out_cnt.at[c, pl.ds(r0, _TAIL)])


def _sc_segcnt(dst_sc, dst_fc):
  mesh = plsc.VectorSubcoreMesh(core_axis_name="c", subcore_axis_name="s",
                                num_cores=_NC, num_subcores=_NS)
  f = pl.kernel(
      _sc_segcnt_body,
      mesh=mesh,
      out_type=jax.ShapeDtypeStruct((_NC, N, F), jnp.float32),
      scratch_types=[
          pltpu.VMEM((_K,), jnp.int32),
          pltpu.VMEM((_K,), jnp.int32),
          pltpu.VMEM((_K, F), jnp.float32),
          pltpu.VMEM((_CZ, F), jnp.float32),
          pltpu.VMEM_SHARED((N, F), jnp.float32),
          pltpu.SemaphoreType.DMA,
          pltpu.SemaphoreType.DMA,
          pltpu.SemaphoreType.DMA,
          pltpu.SemaphoreType.DMA,
      ],
  )
  return f(dst_sc, dst_fc)


_R = 2000
_GRID = N // _R


def _tc_layer_body(batch_ref, x_sc_ref, x_fc_ref, sum_ref, cnt_ref,
                   wrel_sc_ref, wroot_sc_ref, brel_sc_ref,
                   wrel_fc_ref, wroot_fc_ref, brel_fc_ref,
                   h_sc_ref, h_fc_ref, p_sc_ref, p_fc_ref):
  step = pl.program_id(0)
  batch = batch_ref[0, 0, :]
  onehot = (jax.lax.broadcasted_iota(jnp.int32, (G, _R), 0)
            == batch[None, :]).astype(jnp.float32)

  for b, (x_ref, wrel_ref, wroot_ref, brel_ref, h_ref, p_ref) in enumerate([
      (x_sc_ref, wrel_sc_ref, wroot_sc_ref, brel_sc_ref, h_sc_ref, p_sc_ref),
      (x_fc_ref, wrel_fc_ref, wroot_fc_ref, brel_fc_ref, h_fc_ref, p_fc_ref),
  ]):
    cnt = jnp.maximum(cnt_ref[b, :, 0:1], 1.0)
    agg = sum_ref[b] / cnt
    x = x_ref[...]
    h = jnp.dot(agg, wrel_ref[...], preferred_element_type=jnp.float32)
    h = h + jnp.dot(x, wroot_ref[...], preferred_element_type=jnp.float32)
    h = jnp.maximum(h + brel_ref[...][None, :], 0.0)
    h_ref[...] = h
    part = jnp.dot(onehot, h, preferred_element_type=jnp.float32)

    @pl.when(step == 0)
    def _():
      p_ref[...] = part

    @pl.when(step != 0)
    def _():
      p_ref[...] += part


def _tc_layer(batch3, x_sc, x_fc, sums, cnts,
              wrel_sc, wroot_sc, brel_sc, wrel_fc, wroot_fc, brel_fc):
  full = lambda shape: pl.BlockSpec(shape, lambda i: tuple(0 for _ in shape))
  return pl.pallas_call(
      _tc_layer_body,
      grid=(_GRID,),
      in_specs=[
          pl.BlockSpec((1, 1, _R), lambda i: (i, 0, 0)),
          pl.BlockSpec((_R, F), lambda i: (i, 0)),
          pl.BlockSpec((_R, F), lambda i: (i, 0)),
          pl.BlockSpec((_NC, _R, F), lambda i: (0, i, 0)),
          pl.BlockSpec((_NC, _R, F), lambda i: (0, i, 0)),
          full((F, F)), full((F, F)), full((F,)),
          full((F, F)), full((F, F)), full((F,)),
      ],
      out_specs=[
          pl.BlockSpec((_R, F), lambda i: (i, 0)),
          pl.BlockSpec((_R, F), lambda i: (i, 0)),
          pl.BlockSpec((G, F), lambda i: (0, 0)),
          pl.BlockSpec((G, F), lambda i: (0, 0)),
      ],
      out_shape=[
          jax.ShapeDtypeStruct((N, F), jnp.float32),
          jax.ShapeDtypeStruct((N, F), jnp.float32),
          jax.ShapeDtypeStruct((G, F), jnp.float32),
          jax.ShapeDtypeStruct((G, F), jnp.float32),
      ],
  )(batch3, x_sc, x_fc, sums, cnts,
    wrel_sc, wroot_sc, brel_sc, wrel_fc, wroot_fc, brel_fc)


def _tc_head_body(p1s_ref, p2s_ref, p1f_ref, p2f_ref,
                  w1_ref, b1_ref, w2_ref, b2_ref, w3_ref, b3_ref, out_ref):
  xcat = jnp.concatenate(
      [p1s_ref[...], p2s_ref[...], p1f_ref[...], p2f_ref[...]], axis=1)
  x = jnp.maximum(
      jnp.dot(xcat, w1_ref[...], preferred_element_type=jnp.float32)
      + b1_ref[...][None, :], 0.0)
  x = jnp.maximum(
      jnp.dot(x, w2_ref[...], preferred_element_type=jnp.float32)
      + b2_ref[...][None, :], 0.0)
  x = (jnp.dot(x, w3_ref[...], preferred_element_type=jnp.float32)
       + b3_ref[...][None, :])
  m = jnp.max(x, axis=-1, keepdims=True)
  lse = m + jnp.log(jnp.sum(jnp.exp(x - m), axis=-1, keepdims=True))
  out_ref[...] = x - lse


def _tc_head(p1s, p2s, p1f, p2f, w1, b1, w2, b2, w3, b3):
  return pl.pallas_call(
      _tc_head_body,
      out_shape=jax.ShapeDtypeStruct((G, 2), jnp.float32),
  )(p1s, p2s, p1f, p2f, w1, b1, w2, b2, w3, b3)


@jax.jit
def kernel(sc_x, fc_x, sc_edge_index, fc_edge_index, batch,
           sc1_Wrel, sc1_brel, sc1_Wroot, sc2_Wrel, sc2_brel, sc2_Wroot,
           fc1_Wrel, fc1_brel, fc1_Wroot, fc2_Wrel, fc2_brel, fc2_Wroot,
           W1, b1, W2, b2, W3, b3):
  src_sc, dst_sc = sc_edge_index[0], sc_edge_index[1]
  src_fc, dst_fc = fc_edge_index[0], fc_edge_index[1]
  batch3 = batch.reshape(_GRID, 1, _R)

  cnts = _sc_segcnt(dst_sc, dst_fc)
  sums1 = _sc_segsum(sc_x, fc_x, src_sc, dst_sc, src_fc, dst_fc)
  h1_sc, h1_fc, p1_sc, p1_fc = _tc_layer(
      batch3, sc_x, fc_x, sums1, cnts,
      sc1_Wrel, sc1_Wroot, sc1_brel, fc1_Wrel, fc1_Wroot, fc1_brel)

  sums2 = _sc_segsum(h1_sc, h1_fc, src_sc, dst_sc, src_fc, dst_fc)
  h2_sc, h2_fc, p2_sc, p2_fc = _tc_layer(
      batch3, h1_sc, h1_fc, sums2, cnts,
      sc2_Wrel, sc2_Wroot, sc2_brel, fc2_Wrel, fc2_Wroot, fc2_brel)

  return _tc_head(p1_sc, p2_sc, p1_fc, p2_fc, W1, b1, W2, b2, W3, b3)


# speedup vs baseline: 7.1159x; 1.1790x over previous
import functools
import jax
import jax.numpy as jnp
from jax import lax
from jax.experimental import pallas as pl
from jax.experimental.pallas import tpu as pltpu
from jax.experimental.pallas import tpu_sc as plsc

N = 10000
E = 320000
F = 128
G = 64

_NC = 2
_NS = 16
_K = 128
_EP = E // _NS
_NCHUNK = 156
_KT = _EP - _NCHUNK * _K
_TOFF = _NCHUNK * _K
_RT = 624
_CZ = 104
_NCOPY = _RT // _CZ
_TAIL = N - _NS * _RT


def _sc_segsum_body(x_sc, x_fc, src_sc, dst_sc, src_fc, dst_fc,
                    out_sum,
                    idx0, idx1, dst0, dst1, rows0, rows1,
                    idxT, dstT, rowsT,
                    acc_sh, gsem0, gsem1, ssem0, ssem1, isem0, isem1):
  c = lax.axis_index("c")
  s = lax.axis_index("s")

  def _init_rows(i, _):
    def _init_lane(j, _):
      rows0[i, pl.ds(j * 16, 16)] = jnp.zeros((16,), jnp.float32)
      return 0
    lax.fori_loop(0, F // 16, _init_lane, 0)
    return 0
  lax.fori_loop(0, _CZ, _init_rows, 0)

  for j in range(_NCOPY):
    r0 = s * _RT + j * _CZ
    pltpu.sync_copy(rows0.at[pl.ds(0, _CZ)], acc_sh.at[pl.ds(r0, _CZ)])

  @pl.when(s == 0)
  def _():
    pltpu.sync_copy(rows0.at[pl.ds(0, _TAIL)],
                    acc_sh.at[pl.ds(_NS * _RT, _TAIL)])

  plsc.subcore_barrier()

  def _process(x_ref, src_ref, dst_ref):
    bufs = ((idx0, dst0, rows0, gsem0, ssem0, isem0),
            (idx1, dst1, rows1, gsem1, ssem1, isem1))

    def idx_start(chunk, b):
      iv, dv, _, _, _, isem = bufs[b]
      off = s * _EP + chunk * _K
      pltpu.async_copy(src_ref.at[pl.ds(off, _K)], iv, isem)
      pltpu.async_copy(dst_ref.at[pl.ds(off, _K)], dv, isem)

    def idx_wait(b):
      iv, dv, _, _, _, isem = bufs[b]
      pltpu.make_async_copy(src_ref.at[pl.ds(0, _K)], iv, isem).wait()
      pltpu.make_async_copy(dst_ref.at[pl.ds(0, _K)], dv, isem).wait()

    def g_start(b):
      iv, _, rv, gsem, _, _ = bufs[b]
      pltpu.async_copy(x_ref.at[iv], rv, gsem)

    def g_wait(b):
      iv, _, rv, gsem, _, _ = bufs[b]
      pltpu.make_async_copy(x_ref.at[iv], rv, gsem).wait()

    def s_start(b):
      _, dv, rv, _, ssem, _ = bufs[b]
      pltpu.async_copy(rv, acc_sh.at[dv], ssem, add=True)

    def s_wait(b):
      _, dv, rv, _, ssem, _ = bufs[b]
      pltpu.make_async_copy(rv, acc_sh.at[dv], ssem).wait()

    idx_start(0, 0)
    idx_wait(0)
    idx_start(1, 1)
    g_start(0)

    def body(j, _):
      g_wait(0)
      s_start(0)
      idx_wait(1)
      g_start(1)
      s_wait(0)

      @pl.when(2 * j + 2 < _NCHUNK)
      def _():
        idx_start(2 * j + 2, 0)

      g_wait(1)
      s_start(1)

      @pl.when(2 * j + 2 < _NCHUNK)
      def _():
        idx_wait(0)
        g_start(0)

      s_wait(1)

      @pl.when(2 * j + 3 < _NCHUNK)
      def _():
        idx_start(2 * j + 3, 1)

      return 0
    lax.fori_loop(0, _NCHUNK // 2, body, 0)

    offT = s * _EP + _TOFF
    pltpu.sync_copy(src_ref.at[pl.ds(offT, _KT)], idxT)
    pltpu.sync_copy(dst_ref.at[pl.ds(offT, _KT)], dstT)
    pltpu.async_copy(x_ref.at[idxT], rowsT, gsem0).wait()
    pltpu.sync_copy(rowsT, acc_sh.at[dstT], add=True)

  @pl.when(c == 0)
  def _():
    _process(x_sc, src_sc, dst_sc)

  @pl.when(c == 1)
  def _():
    _process(x_fc, src_fc, dst_fc)

  plsc.subcore_barrier()

  for j in range(_NCOPY):
    r0 = s * _RT + j * _CZ
    pltpu.sync_copy(acc_sh.at[pl.ds(r0, _CZ)], rows0.at[pl.ds(0, _CZ)])
    pltpu.sync_copy(rows0.at[pl.ds(0, _CZ)], out_sum.at[c, pl.ds(r0, _CZ)])

  @pl.when(s == 0)
  def _():
    r0 = _NS * _RT
    pltpu.sync_copy(acc_sh.at[pl.ds(r0, _TAIL)], rows0.at[pl.ds(0, _TAIL)])
    pltpu.sync_copy(rows0.at[pl.ds(0, _TAIL)], out_sum.at[c, pl.ds(r0, _TAIL)])


def _sc_segsum(x_sc, x_fc, src_sc, dst_sc, src_fc, dst_fc):
  mesh = plsc.VectorSubcoreMesh(core_axis_name="c", subcore_axis_name="s",
                                num_cores=_NC, num_subcores=_NS)
  f = pl.kernel(
      _sc_segsum_body,
      mesh=mesh,
      out_type=jax.ShapeDtypeStruct((_NC, N, F), jnp.float32),
      scratch_types=[
          pltpu.VMEM((_K,), jnp.int32),
          pltpu.VMEM((_K,), jnp.int32),
          pltpu.VMEM((_K,), jnp.int32),
          pltpu.VMEM((_K,), jnp.int32),
          pltpu.VMEM((_K, F), jnp.float32),
          pltpu.VMEM((_K, F), jnp.float32),
          pltpu.VMEM((_KT,), jnp.int32),
          pltpu.VMEM((_KT,), jnp.int32),
          pltpu.VMEM((_KT, F), jnp.float32),
          pltpu.VMEM_SHARED((N, F), jnp.float32),
          pltpu.SemaphoreType.DMA,
          pltpu.SemaphoreType.DMA,
          pltpu.SemaphoreType.DMA,
          pltpu.SemaphoreType.DMA,
          pltpu.SemaphoreType.DMA,
          pltpu.SemaphoreType.DMA,
      ],
  )
  return f(x_sc, x_fc, src_sc, dst_sc, src_fc, dst_fc)


def _sc_segcnt_body(dst_sc, dst_fc, out_cnt,
                    dst0, dst1, dstT, ones_v, zcnt_v, cnt_sh,
                    ssem0, ssem1, isem0, isem1):
  c = lax.axis_index("c")
  s = lax.axis_index("s")

  def _init_rows(i, _):
    def _init_lane(j, _):
      zcnt_v[i, pl.ds(j * 16, 16)] = jnp.zeros((16,), jnp.float32)
      return 0
    lax.fori_loop(0, F // 16, _init_lane, 0)
    return 0
  lax.fori_loop(0, _CZ, _init_rows, 0)

  def _init_ones(i, _):
    def _init_lane(j, _):
      ones_v[i, pl.ds(j * 16, 16)] = jnp.ones((16,), jnp.float32)
      return 0
    lax.fori_loop(0, F // 16, _init_lane, 0)
    return 0
  lax.fori_loop(0, _K, _init_ones, 0)

  for j in range(_NCOPY):
    r0 = s * _RT + j * _CZ
    pltpu.sync_copy(zcnt_v, cnt_sh.at[pl.ds(r0, _CZ)])

  @pl.when(s == 0)
  def _():
    pltpu.sync_copy(zcnt_v.at[pl.ds(0, _TAIL)],
                    cnt_sh.at[pl.ds(_NS * _RT, _TAIL)])

  plsc.subcore_barrier()

  def _process(dst_ref):
    bufs = ((dst0, ssem0, isem0), (dst1, ssem1, isem1))

    def idx_start(chunk, b):
      dv, _, isem = bufs[b]
      off = s * _EP + chunk * _K
      pltpu.async_copy(dst_ref.at[pl.ds(off, _K)], dv, isem)

    def idx_wait(b):
      dv, _, isem = bufs[b]
      pltpu.make_async_copy(dst_ref.at[pl.ds(0, _K)], dv, isem).wait()

    def s_start(b):
      dv, ssem, _ = bufs[b]
      pltpu.async_copy(ones_v, cnt_sh.at[dv], ssem, add=True)

    def s_wait(b):
      dv, ssem, _ = bufs[b]
      pltpu.make_async_copy(ones_v, cnt_sh.at[dv], ssem).wait()

    idx_start(0, 0)
    idx_start(1, 1)

    def body(j, _):
      idx_wait(0)
      s_start(0)
      idx_wait(1)
      s_start(1)
      s_wait(0)

      @pl.when(2 * j + 2 < _NCHUNK)
      def _():
        idx_start(2 * j + 2, 0)

      s_wait(1)

      @pl.when(2 * j + 3 < _NCHUNK)
      def _():
        idx_start(2 * j + 3, 1)

      return 0
    lax.fori_loop(0, _NCHUNK // 2, body, 0)

    offT = s * _EP + _TOFF
    pltpu.sync_copy(dst_ref.at[pl.ds(offT, _KT)], dstT)
    pltpu.sync_copy(ones_v.at[pl.ds(0, _KT)], cnt_sh.at[dstT], add=True)

  @pl.when(c == 0)
  def _():
    _process(dst_sc)

  @pl.when(c == 1)
  def _():
    _process(dst_fc)

  plsc.subcore_barrier()

  for j in range(_NCOPY):
    r0 = s * _RT + j * _CZ
    pltpu.sync_copy(cnt_sh.at[pl.ds(r0, _CZ)], zcnt_v)
    pltpu.sync_copy(zcnt_v, out_cnt.at[c, pl.ds(r0, _CZ)])

  @pl.when(s == 0)
  def _():
    r0 = _NS * _RT
    pltpu.sync_copy(cnt_sh.at[pl.ds(r0, _TAIL)], zcnt_v.at[pl.ds(0, _TAIL)])
    pltpu.sync_copy(zcnt_v.at[pl.ds(0, _TAIL)], out_cnt.at[c, pl.ds(r0, _TAIL)])


def _sc_segcnt(dst_sc, dst_fc):
  mesh = plsc.VectorSubcoreMesh(core_axis_name="c", subcore_axis_name="s",
                                num_cores=_NC, num_subcores=_NS)
  f = pl.kernel(
      _sc_segcnt_body,
      mesh=mesh,
      out_type=jax.ShapeDtypeStruct((_NC, N, F), jnp.float32),
      scratch_types=[
          pltpu.VMEM((_K,), jnp.int32),
          pltpu.VMEM((_K,), jnp.int32),
          pltpu.VMEM((_KT,), jnp.int32),
          pltpu.VMEM((_K, F), jnp.float32),
          pltpu.VMEM((_CZ, F), jnp.float32),
          pltpu.VMEM_SHARED((N, F), jnp.float32),
          pltpu.SemaphoreType.DMA,
          pltpu.SemaphoreType.DMA,
          pltpu.SemaphoreType.DMA,
          pltpu.SemaphoreType.DMA,
      ],
  )
  return f(dst_sc, dst_fc)


_R = 2000
_GRID = N // _R


def _tc_layer_body(batch_ref, x_sc_ref, x_fc_ref, sum_ref, cnt_ref,
                   wrel_sc_ref, wroot_sc_ref, brel_sc_ref,
                   wrel_fc_ref, wroot_fc_ref, brel_fc_ref,
                   h_sc_ref, h_fc_ref, p_sc_ref, p_fc_ref):
  step = pl.program_id(0)
  batch = batch_ref[0, 0, :]
  onehot = (jax.lax.broadcasted_iota(jnp.int32, (G, _R), 0)
            == batch[None, :]).astype(jnp.float32)

  for b, (x_ref, wrel_ref, wroot_ref, brel_ref, h_ref, p_ref) in enumerate([
      (x_sc_ref, wrel_sc_ref, wroot_sc_ref, brel_sc_ref, h_sc_ref, p_sc_ref),
      (x_fc_ref, wrel_fc_ref, wroot_fc_ref, brel_fc_ref, h_fc_ref, p_fc_ref),
  ]):
    cnt = jnp.maximum(cnt_ref[b, :, 0:1], 1.0)
    agg = sum_ref[b] / cnt
    x = x_ref[...]
    h = jnp.dot(agg, wrel_ref[...], preferred_element_type=jnp.float32)
    h = h + jnp.dot(x, wroot_ref[...], preferred_element_type=jnp.float32)
    h = jnp.maximum(h + brel_ref[...][None, :], 0.0)
    h_ref[...] = h
    part = jnp.dot(onehot, h, preferred_element_type=jnp.float32)

    @pl.when(step == 0)
    def _():
      p_ref[...] = part

    @pl.when(step != 0)
    def _():
      p_ref[...] += part


def _tc_layer(batch3, x_sc, x_fc, sums, cnts,
              wrel_sc, wroot_sc, brel_sc, wrel_fc, wroot_fc, brel_fc):
  full = lambda shape: pl.BlockSpec(shape, lambda i: tuple(0 for _ in shape))
  return pl.pallas_call(
      _tc_layer_body,
      grid=(_GRID,),
      in_specs=[
          pl.BlockSpec((1, 1, _R), lambda i: (i, 0, 0)),
          pl.BlockSpec((_R, F), lambda i: (i, 0)),
          pl.BlockSpec((_R, F), lambda i: (i, 0)),
          pl.BlockSpec((_NC, _R, F), lambda i: (0, i, 0)),
          pl.BlockSpec((_NC, _R, F), lambda i: (0, i, 0)),
          full((F, F)), full((F, F)), full((F,)),
          full((F, F)), full((F, F)), full((F,)),
      ],
      out_specs=[
          pl.BlockSpec((_R, F), lambda i: (i, 0)),
          pl.BlockSpec((_R, F), lambda i: (i, 0)),
          pl.BlockSpec((G, F), lambda i: (0, 0)),
          pl.BlockSpec((G, F), lambda i: (0, 0)),
      ],
      out_shape=[
          jax.ShapeDtypeStruct((N, F), jnp.float32),
          jax.ShapeDtypeStruct((N, F), jnp.float32),
          jax.ShapeDtypeStruct((G, F), jnp.float32),
          jax.ShapeDtypeStruct((G, F), jnp.float32),
      ],
  )(batch3, x_sc, x_fc, sums, cnts,
    wrel_sc, wroot_sc, brel_sc, wrel_fc, wroot_fc, brel_fc)


def _tc_head_body(p1s_ref, p2s_ref, p1f_ref, p2f_ref,
                  w1_ref, b1_ref, w2_ref, b2_ref, w3_ref, b3_ref, out_ref):
  xcat = jnp.concatenate(
      [p1s_ref[...], p2s_ref[...], p1f_ref[...], p2f_ref[...]], axis=1)
  x = jnp.maximum(
      jnp.dot(xcat, w1_ref[...], preferred_element_type=jnp.float32)
      + b1_ref[...][None, :], 0.0)
  x = jnp.maximum(
      jnp.dot(x, w2_ref[...], preferred_element_type=jnp.float32)
      + b2_ref[...][None, :], 0.0)
  x = (jnp.dot(x, w3_ref[...], preferred_element_type=jnp.float32)
       + b3_ref[...][None, :])
  m = jnp.max(x, axis=-1, keepdims=True)
  lse = m + jnp.log(jnp.sum(jnp.exp(x - m), axis=-1, keepdims=True))
  out_ref[...] = x - lse


def _tc_head(p1s, p2s, p1f, p2f, w1, b1, w2, b2, w3, b3):
  return pl.pallas_call(
      _tc_head_body,
      out_shape=jax.ShapeDtypeStruct((G, 2), jnp.float32),
  )(p1s, p2s, p1f, p2f, w1, b1, w2, b2, w3, b3)


@jax.jit
def kernel(sc_x, fc_x, sc_edge_index, fc_edge_index, batch,
           sc1_Wrel, sc1_brel, sc1_Wroot, sc2_Wrel, sc2_brel, sc2_Wroot,
           fc1_Wrel, fc1_brel, fc1_Wroot, fc2_Wrel, fc2_brel, fc2_Wroot,
           W1, b1, W2, b2, W3, b3):
  src_sc, dst_sc = sc_edge_index[0], sc_edge_index[1]
  src_fc, dst_fc = fc_edge_index[0], fc_edge_index[1]
  batch3 = batch.reshape(_GRID, 1, _R)

  cnts = _sc_segcnt(dst_sc, dst_fc)
  sums1 = _sc_segsum(sc_x, fc_x, src_sc, dst_sc, src_fc, dst_fc)
  h1_sc, h1_fc, p1_sc, p1_fc = _tc_layer(
      batch3, sc_x, fc_x, sums1, cnts,
      sc1_Wrel, sc1_Wroot, sc1_brel, fc1_Wrel, fc1_Wroot, fc1_brel)

  sums2 = _sc_segsum(h1_sc, h1_fc, src_sc, dst_sc, src_fc, dst_fc)
  h2_sc, h2_fc, p2_sc, p2_fc = _tc_layer(
      batch3, h1_sc, h1_fc, sums2, cnts,
      sc2_Wrel, sc2_Wroot, sc2_brel, fc2_Wrel, fc2_Wroot, fc2_brel)

  return _tc_head(p1_sc, p2_sc, p1_fc, p2_fc, W1, b1, W2, b2, W3, b3)


# speedup vs baseline: 8.6795x; 1.2197x over previous
import functools
import jax
import jax.numpy as jnp
from jax import lax
from jax.experimental import pallas as pl
from jax.experimental.pallas import tpu as pltpu
from jax.experimental.pallas import tpu_sc as plsc

N = 10000
E = 320000
F = 128
G = 64

_NC = 2
_NS = 16
_K = 128
_EP = E // _NS
_NCHUNK = 156
_KT = _EP - _NCHUNK * _K
_TOFF = _NCHUNK * _K
_RT = 624
_CZ = 104
_NCOPY = _RT // _CZ
_TAIL = N - _NS * _RT


def _sc_segsum_body(x_sc, x_fc, src_sc, dst_sc, src_fc, dst_fc,
                    out_sum, out_cnt,
                    idx0, idx1, dst0, dst1, rows0, rows1,
                    idxT, dstT, rowsT, zc1_v, ones1_v,
                    acc_sh, cnt_sh,
                    gsem0, gsem1, ssem0, ssem1, isem0, isem1, csem0, csem1):
  c = lax.axis_index("c")
  s = lax.axis_index("s")

  def _init_rows(i, _):
    def _init_lane(j, _):
      rows0[i, pl.ds(j * 16, 16)] = jnp.zeros((16,), jnp.float32)
      return 0
    lax.fori_loop(0, F // 16, _init_lane, 0)
    return 0
  lax.fori_loop(0, _CZ, _init_rows, 0)

  def _init_1d(i, _):
    zc1_v[pl.ds(i * 16, 16)] = jnp.zeros((16,), jnp.float32)
    return 0
  lax.fori_loop(0, _RT // 16, _init_1d, 0)

  def _init_ones(i, _):
    ones1_v[pl.ds(i * 16, 16)] = jnp.ones((16,), jnp.float32)
    return 0
  lax.fori_loop(0, _K // 16, _init_ones, 0)

  for j in range(_NCOPY):
    r0 = s * _RT + j * _CZ
    pltpu.sync_copy(rows0.at[pl.ds(0, _CZ)], acc_sh.at[pl.ds(r0, _CZ)])
  pltpu.sync_copy(zc1_v, cnt_sh.at[pl.ds(s * _RT, _RT)])

  @pl.when(s == 0)
  def _():
    pltpu.sync_copy(rows0.at[pl.ds(0, _TAIL)],
                    acc_sh.at[pl.ds(_NS * _RT, _TAIL)])
    pltpu.sync_copy(zc1_v.at[pl.ds(0, _TAIL)],
                    cnt_sh.at[pl.ds(_NS * _RT, _TAIL)])

  plsc.subcore_barrier()

  def _process(x_ref, src_ref, dst_ref):
    bufs = ((idx0, dst0, rows0, gsem0, ssem0, isem0, csem0),
            (idx1, dst1, rows1, gsem1, ssem1, isem1, csem1))

    def idx_start(chunk, b):
      iv, dv, _, _, _, isem, _ = bufs[b]
      off = s * _EP + chunk * _K
      pltpu.async_copy(src_ref.at[pl.ds(off, _K)], iv, isem)
      pltpu.async_copy(dst_ref.at[pl.ds(off, _K)], dv, isem)

    def idx_wait(b):
      iv, dv, _, _, _, isem, _ = bufs[b]
      pltpu.make_async_copy(src_ref.at[pl.ds(0, _K)], iv, isem).wait()
      pltpu.make_async_copy(dst_ref.at[pl.ds(0, _K)], dv, isem).wait()

    def g_start(b):
      iv, _, rv, gsem, _, _, _ = bufs[b]
      pltpu.async_copy(x_ref.at[iv], rv, gsem)

    def g_wait(b):
      iv, _, rv, gsem, _, _, _ = bufs[b]
      pltpu.make_async_copy(x_ref.at[iv], rv, gsem).wait()

    def s_start(b):
      _, dv, rv, _, ssem, _, csem = bufs[b]
      pltpu.async_copy(rv, acc_sh.at[dv], ssem, add=True)
      pltpu.async_copy(ones1_v, cnt_sh.at[dv], csem, add=True)

    def s_wait(b):
      _, dv, rv, _, ssem, _, csem = bufs[b]
      pltpu.make_async_copy(rv, acc_sh.at[dv], ssem).wait()
      pltpu.make_async_copy(ones1_v, cnt_sh.at[dv], csem).wait()

    idx_start(0, 0)
    idx_wait(0)
    idx_start(1, 1)
    g_start(0)

    def body(j, _):
      g_wait(0)
      s_start(0)
      idx_wait(1)
      g_start(1)
      s_wait(0)

      @pl.when(2 * j + 2 < _NCHUNK)
      def _():
        idx_start(2 * j + 2, 0)

      g_wait(1)
      s_start(1)

      @pl.when(2 * j + 2 < _NCHUNK)
      def _():
        idx_wait(0)
        g_start(0)

      s_wait(1)

      @pl.when(2 * j + 3 < _NCHUNK)
      def _():
        idx_start(2 * j + 3, 1)

      return 0
    lax.fori_loop(0, _NCHUNK // 2, body, 0)

    offT = s * _EP + _TOFF
    pltpu.sync_copy(src_ref.at[pl.ds(offT, _KT)], idxT)
    pltpu.sync_copy(dst_ref.at[pl.ds(offT, _KT)], dstT)
    pltpu.async_copy(x_ref.at[idxT], rowsT, gsem0).wait()
    pltpu.sync_copy(rowsT, acc_sh.at[dstT], add=True)
    pltpu.sync_copy(ones1_v.at[pl.ds(0, _KT)], cnt_sh.at[dstT], add=True)

  @pl.when(c == 0)
  def _():
    _process(x_sc, src_sc, dst_sc)

  @pl.when(c == 1)
  def _():
    _process(x_fc, src_fc, dst_fc)

  plsc.subcore_barrier()

  for j in range(_NCOPY):
    r0 = s * _RT + j * _CZ
    pltpu.sync_copy(acc_sh.at[pl.ds(r0, _CZ)], rows0.at[pl.ds(0, _CZ)])
    pltpu.sync_copy(rows0.at[pl.ds(0, _CZ)], out_sum.at[c, pl.ds(r0, _CZ)])
  pltpu.sync_copy(cnt_sh.at[pl.ds(s * _RT, _RT)], zc1_v)
  pltpu.sync_copy(zc1_v, out_cnt.at[pl.ds(c * N + s * _RT, _RT)])

  @pl.when(s == 0)
  def _():
    r0 = _NS * _RT
    pltpu.sync_copy(acc_sh.at[pl.ds(r0, _TAIL)], rows0.at[pl.ds(0, _TAIL)])
    pltpu.sync_copy(rows0.at[pl.ds(0, _TAIL)], out_sum.at[c, pl.ds(r0, _TAIL)])
    pltpu.sync_copy(cnt_sh.at[pl.ds(r0, _TAIL)], zc1_v.at[pl.ds(0, _TAIL)])
    pltpu.sync_copy(zc1_v.at[pl.ds(0, _TAIL)], out_cnt.at[pl.ds(c * N + r0, _TAIL)])


def _sc_segsum(x_sc, x_fc, src_sc, dst_sc, src_fc, dst_fc):
  mesh = plsc.VectorSubcoreMesh(core_axis_name="c", subcore_axis_name="s",
                                num_cores=_NC, num_subcores=_NS)
  f = pl.kernel(
      _sc_segsum_body,
      mesh=mesh,
      out_type=[
          jax.ShapeDtypeStruct((_NC, N, F), jnp.float32),
          jax.ShapeDtypeStruct((_NC * N,), jnp.float32),
      ],
      scratch_types=[
          pltpu.VMEM((_K,), jnp.int32),
          pltpu.VMEM((_K,), jnp.int32),
          pltpu.VMEM((_K,), jnp.int32),
          pltpu.VMEM((_K,), jnp.int32),
          pltpu.VMEM((_K, F), jnp.float32),
          pltpu.VMEM((_K, F), jnp.float32),
          pltpu.VMEM((_KT,), jnp.int32),
          pltpu.VMEM((_KT,), jnp.int32),
          pltpu.VMEM((_KT, F), jnp.float32),
          pltpu.VMEM((_RT,), jnp.float32),
          pltpu.VMEM((_K,), jnp.float32),
          pltpu.VMEM_SHARED((N, F), jnp.float32),
          pltpu.VMEM_SHARED((N,), jnp.float32),
          pltpu.SemaphoreType.DMA,
          pltpu.SemaphoreType.DMA,
          pltpu.SemaphoreType.DMA,
          pltpu.SemaphoreType.DMA,
          pltpu.SemaphoreType.DMA,
          pltpu.SemaphoreType.DMA,
          pltpu.SemaphoreType.DMA,
          pltpu.SemaphoreType.DMA,
      ],
  )
  return f(x_sc, x_fc, src_sc, dst_sc, src_fc, dst_fc)


_R = 2000
_GRID = N // _R


def _tc_layer_body(batch_ref, x_sc_ref, x_fc_ref, sum_ref, cnt_ref,
                   wrel_sc_ref, wroot_sc_ref, brel_sc_ref,
                   wrel_fc_ref, wroot_fc_ref, brel_fc_ref,
                   h_sc_ref, h_fc_ref, p_sc_ref, p_fc_ref):
  step = pl.program_id(0)
  batch = batch_ref[0, 0, :]
  onehot = (jax.lax.broadcasted_iota(jnp.int32, (G, _R), 0)
            == batch[None, :]).astype(jnp.float32)

  for b, (x_ref, wrel_ref, wroot_ref, brel_ref, h_ref, p_ref) in enumerate([
      (x_sc_ref, wrel_sc_ref, wroot_sc_ref, brel_sc_ref, h_sc_ref, p_sc_ref),
      (x_fc_ref, wrel_fc_ref, wroot_fc_ref, brel_fc_ref, h_fc_ref, p_fc_ref),
  ]):
    cnt = jnp.maximum(cnt_ref[b, :, 0], 1.0)[:, None]
    agg = sum_ref[b] / cnt
    x = x_ref[...]
    h = jnp.dot(agg, wrel_ref[...], preferred_element_type=jnp.float32)
    h = h + jnp.dot(x, wroot_ref[...], preferred_element_type=jnp.float32)
    h = jnp.maximum(h + brel_ref[...][None, :], 0.0)
    h_ref[...] = h
    part = jnp.dot(onehot, h, preferred_element_type=jnp.float32)

    @pl.when(step == 0)
    def _():
      p_ref[...] = part

    @pl.when(step != 0)
    def _():
      p_ref[...] += part


def _tc_layer(batch3, x_sc, x_fc, sums, cnts,
              wrel_sc, wroot_sc, brel_sc, wrel_fc, wroot_fc, brel_fc):
  full = lambda shape: pl.BlockSpec(shape, lambda i: tuple(0 for _ in shape))
  return pl.pallas_call(
      _tc_layer_body,
      grid=(_GRID,),
      in_specs=[
          pl.BlockSpec((1, 1, _R), lambda i: (i, 0, 0)),
          pl.BlockSpec((_R, F), lambda i: (i, 0)),
          pl.BlockSpec((_R, F), lambda i: (i, 0)),
          pl.BlockSpec((_NC, _R, F), lambda i: (0, i, 0)),
          pl.BlockSpec((_NC, _R, 1), lambda i: (0, i, 0)),
          full((F, F)), full((F, F)), full((F,)),
          full((F, F)), full((F, F)), full((F,)),
      ],
      out_specs=[
          pl.BlockSpec((_R, F), lambda i: (i, 0)),
          pl.BlockSpec((_R, F), lambda i: (i, 0)),
          pl.BlockSpec((G, F), lambda i: (0, 0)),
          pl.BlockSpec((G, F), lambda i: (0, 0)),
      ],
      out_shape=[
          jax.ShapeDtypeStruct((N, F), jnp.float32),
          jax.ShapeDtypeStruct((N, F), jnp.float32),
          jax.ShapeDtypeStruct((G, F), jnp.float32),
          jax.ShapeDtypeStruct((G, F), jnp.float32),
      ],
  )(batch3, x_sc, x_fc, sums, cnts,
    wrel_sc, wroot_sc, brel_sc, wrel_fc, wroot_fc, brel_fc)


def _tc_head_body(p1s_ref, p2s_ref, p1f_ref, p2f_ref,
                  w1_ref, b1_ref, w2_ref, b2_ref, w3_ref, b3_ref, out_ref):
  xcat = jnp.concatenate(
      [p1s_ref[...], p2s_ref[...], p1f_ref[...], p2f_ref[...]], axis=1)
  x = jnp.maximum(
      jnp.dot(xcat, w1_ref[...], preferred_element_type=jnp.float32)
      + b1_ref[...][None, :], 0.0)
  x = jnp.maximum(
      jnp.dot(x, w2_ref[...], preferred_element_type=jnp.float32)
      + b2_ref[...][None, :], 0.0)
  x = (jnp.dot(x, w3_ref[...], preferred_element_type=jnp.float32)
       + b3_ref[...][None, :])
  m = jnp.max(x, axis=-1, keepdims=True)
  lse = m + jnp.log(jnp.sum(jnp.exp(x - m), axis=-1, keepdims=True))
  out_ref[...] = x - lse


def _tc_head(p1s, p2s, p1f, p2f, w1, b1, w2, b2, w3, b3):
  return pl.pallas_call(
      _tc_head_body,
      out_shape=jax.ShapeDtypeStruct((G, 2), jnp.float32),
  )(p1s, p2s, p1f, p2f, w1, b1, w2, b2, w3, b3)


@jax.jit
def kernel(sc_x, fc_x, sc_edge_index, fc_edge_index, batch,
           sc1_Wrel, sc1_brel, sc1_Wroot, sc2_Wrel, sc2_brel, sc2_Wroot,
           fc1_Wrel, fc1_brel, fc1_Wroot, fc2_Wrel, fc2_brel, fc2_Wroot,
           W1, b1, W2, b2, W3, b3):
  src_sc, dst_sc = sc_edge_index[0], sc_edge_index[1]
  src_fc, dst_fc = fc_edge_index[0], fc_edge_index[1]
  batch3 = batch.reshape(_GRID, 1, _R)

  sums1, cnt_flat = _sc_segsum(sc_x, fc_x, src_sc, dst_sc, src_fc, dst_fc)
  cnts = cnt_flat.reshape(_NC, N, 1)
  h1_sc, h1_fc, p1_sc, p1_fc = _tc_layer(
      batch3, sc_x, fc_x, sums1, cnts,
      sc1_Wrel, sc1_Wroot, sc1_brel, fc1_Wrel, fc1_Wroot, fc1_brel)

  sums2, _ = _sc_segsum(h1_sc, h1_fc, src_sc, dst_sc, src_fc, dst_fc)
  h2_sc, h2_fc, p2_sc, p2_fc = _tc_layer(
      batch3, h1_sc, h1_fc, sums2, cnts,
      sc2_Wrel, sc2_Wroot, sc2_brel, fc2_Wrel, fc2_Wroot, fc2_brel)

  return _tc_head(p1_sc, p2_sc, p1_fc, p2_fc, W1, b1, W2, b2, W3, b3)


# speedup vs baseline: 8.8136x; 1.0155x over previous
import functools
import jax
import jax.numpy as jnp
from jax import lax
from jax.experimental import pallas as pl
from jax.experimental.pallas import tpu as pltpu
from jax.experimental.pallas import tpu_sc as plsc

N = 10000
E = 320000
F = 128
G = 64

_NC = 2
_NS = 16
_K = 128
_EP = E // _NS
_NCHUNK = 156
_KT = _EP - _NCHUNK * _K
_TOFF = _NCHUNK * _K
_RT = 624
_CZ = 104
_NCOPY = _RT // _CZ
_TAIL = N - _NS * _RT


def _sc_segsum_body(with_cnt,
                    x_sc, x_fc, src_sc, dst_sc, src_fc, dst_fc,
                    out_sum, out_cnt,
                    idx0, idx1, dst0, dst1, rows0, rows1,
                    idxT, dstT, rowsT, zc1_v, ones1_v,
                    acc_sh, cnt_sh,
                    gsem0, gsem1, ssem0, ssem1, isem0, isem1, csem0, csem1):
  c = lax.axis_index("c")
  s = lax.axis_index("s")

  def _init_rows(i, _):
    def _init_lane(j, _):
      rows0[i, pl.ds(j * 16, 16)] = jnp.zeros((16,), jnp.float32)
      return 0
    lax.fori_loop(0, F // 16, _init_lane, 0)
    return 0
  lax.fori_loop(0, _CZ, _init_rows, 0)

  def _init_1d(i, _):
    zc1_v[pl.ds(i * 16, 16)] = jnp.zeros((16,), jnp.float32)
    return 0
  lax.fori_loop(0, _RT // 16, _init_1d, 0)

  if with_cnt:
    def _init_ones(i, _):
      ones1_v[pl.ds(i * 16, 16)] = jnp.ones((16,), jnp.float32)
      return 0
    lax.fori_loop(0, _K // 16, _init_ones, 0)

  for j in range(_NCOPY):
    r0 = s * _RT + j * _CZ
    pltpu.sync_copy(rows0.at[pl.ds(0, _CZ)], acc_sh.at[pl.ds(r0, _CZ)])
  if with_cnt:
    pltpu.sync_copy(zc1_v, cnt_sh.at[pl.ds(s * _RT, _RT)])

  @pl.when(s == 0)
  def _():
    pltpu.sync_copy(rows0.at[pl.ds(0, _TAIL)],
                    acc_sh.at[pl.ds(_NS * _RT, _TAIL)])
    if with_cnt:
      pltpu.sync_copy(zc1_v.at[pl.ds(0, _TAIL)],
                      cnt_sh.at[pl.ds(_NS * _RT, _TAIL)])

  plsc.subcore_barrier()

  def _process(x_ref, src_ref, dst_ref):
    bufs = ((idx0, dst0, rows0, gsem0, ssem0, isem0, csem0),
            (idx1, dst1, rows1, gsem1, ssem1, isem1, csem1))

    def idx_start(chunk, b):
      iv, dv, _, _, _, isem, _ = bufs[b]
      off = s * _EP + chunk * _K
      pltpu.async_copy(src_ref.at[pl.ds(off, _K)], iv, isem)
      pltpu.async_copy(dst_ref.at[pl.ds(off, _K)], dv, isem)

    def idx_wait(b):
      iv, dv, _, _, _, isem, _ = bufs[b]
      pltpu.make_async_copy(src_ref.at[pl.ds(0, _K)], iv, isem).wait()
      pltpu.make_async_copy(dst_ref.at[pl.ds(0, _K)], dv, isem).wait()

    def g_start(b):
      iv, _, rv, gsem, _, _, _ = bufs[b]
      pltpu.async_copy(x_ref.at[iv], rv, gsem)

    def g_wait(b):
      iv, _, rv, gsem, _, _, _ = bufs[b]
      pltpu.make_async_copy(x_ref.at[iv], rv, gsem).wait()

    def s_start(b):
      _, dv, rv, _, ssem, _, csem = bufs[b]
      pltpu.async_copy(rv, acc_sh.at[dv], ssem, add=True)
      if with_cnt:
        pltpu.async_copy(ones1_v, cnt_sh.at[dv], csem, add=True)

    def s_wait(b):
      _, dv, rv, _, ssem, _, csem = bufs[b]
      pltpu.make_async_copy(rv, acc_sh.at[dv], ssem).wait()
      if with_cnt:
        pltpu.make_async_copy(ones1_v, cnt_sh.at[dv], csem).wait()

    idx_start(0, 0)
    idx_wait(0)
    idx_start(1, 1)
    g_start(0)

    def body(j, _):
      g_wait(0)
      s_start(0)
      idx_wait(1)
      g_start(1)
      s_wait(0)

      @pl.when(2 * j + 2 < _NCHUNK)
      def _():
        idx_start(2 * j + 2, 0)

      g_wait(1)
      s_start(1)

      @pl.when(2 * j + 2 < _NCHUNK)
      def _():
        idx_wait(0)
        g_start(0)

      s_wait(1)

      @pl.when(2 * j + 3 < _NCHUNK)
      def _():
        idx_start(2 * j + 3, 1)

      return 0
    lax.fori_loop(0, _NCHUNK // 2, body, 0)

    offT = s * _EP + _TOFF
    pltpu.sync_copy(src_ref.at[pl.ds(offT, _KT)], idxT)
    pltpu.sync_copy(dst_ref.at[pl.ds(offT, _KT)], dstT)
    pltpu.async_copy(x_ref.at[idxT], rowsT, gsem0).wait()
    pltpu.sync_copy(rowsT, acc_sh.at[dstT], add=True)
    if with_cnt:
      pltpu.sync_copy(ones1_v.at[pl.ds(0, _KT)], cnt_sh.at[dstT], add=True)

  @pl.when(c == 0)
  def _():
    _process(x_sc, src_sc, dst_sc)

  @pl.when(c == 1)
  def _():
    _process(x_fc, src_fc, dst_fc)

  plsc.subcore_barrier()

  for j in range(_NCOPY):
    r0 = s * _RT + j * _CZ
    pltpu.sync_copy(acc_sh.at[pl.ds(r0, _CZ)], rows0.at[pl.ds(0, _CZ)])
    pltpu.sync_copy(rows0.at[pl.ds(0, _CZ)], out_sum.at[c, pl.ds(r0, _CZ)])
  if with_cnt:
    pltpu.sync_copy(cnt_sh.at[pl.ds(s * _RT, _RT)], zc1_v)
    pltpu.sync_copy(zc1_v, out_cnt.at[pl.ds(c * N + s * _RT, _RT)])

  @pl.when(s == 0)
  def _():
    r0 = _NS * _RT
    pltpu.sync_copy(acc_sh.at[pl.ds(r0, _TAIL)], rows0.at[pl.ds(0, _TAIL)])
    pltpu.sync_copy(rows0.at[pl.ds(0, _TAIL)], out_sum.at[c, pl.ds(r0, _TAIL)])
    if with_cnt:
      pltpu.sync_copy(cnt_sh.at[pl.ds(r0, _TAIL)], zc1_v.at[pl.ds(0, _TAIL)])
      pltpu.sync_copy(zc1_v.at[pl.ds(0, _TAIL)],
                      out_cnt.at[pl.ds(c * N + r0, _TAIL)])


def _sc_segsum(x_sc, x_fc, src_sc, dst_sc, src_fc, dst_fc, with_cnt=True):
  mesh = plsc.VectorSubcoreMesh(core_axis_name="c", subcore_axis_name="s",
                                num_cores=_NC, num_subcores=_NS)
  f = pl.kernel(
      functools.partial(_sc_segsum_body, with_cnt),
      mesh=mesh,
      out_type=[
          jax.ShapeDtypeStruct((_NC, N, F), jnp.float32),
          jax.ShapeDtypeStruct((_NC * N,), jnp.float32),
      ],
      scratch_types=[
          pltpu.VMEM((_K,), jnp.int32),
          pltpu.VMEM((_K,), jnp.int32),
          pltpu.VMEM((_K,), jnp.int32),
          pltpu.VMEM((_K,), jnp.int32),
          pltpu.VMEM((_K, F), jnp.float32),
          pltpu.VMEM((_K, F), jnp.float32),
          pltpu.VMEM((_KT,), jnp.int32),
          pltpu.VMEM((_KT,), jnp.int32),
          pltpu.VMEM((_KT, F), jnp.float32),
          pltpu.VMEM((_RT,), jnp.float32),
          pltpu.VMEM((_K,), jnp.float32),
          pltpu.VMEM_SHARED((N, F), jnp.float32),
          pltpu.VMEM_SHARED((N,), jnp.float32),
          pltpu.SemaphoreType.DMA,
          pltpu.SemaphoreType.DMA,
          pltpu.SemaphoreType.DMA,
          pltpu.SemaphoreType.DMA,
          pltpu.SemaphoreType.DMA,
          pltpu.SemaphoreType.DMA,
          pltpu.SemaphoreType.DMA,
          pltpu.SemaphoreType.DMA,
      ],
  )
  return f(x_sc, x_fc, src_sc, dst_sc, src_fc, dst_fc)


_R = 2000
_GRID = N // _R


def _tc_layer_body(batch_ref, x_sc_ref, x_fc_ref, sum_ref, cnt_ref,
                   wrel_sc_ref, wroot_sc_ref, brel_sc_ref,
                   wrel_fc_ref, wroot_fc_ref, brel_fc_ref,
                   h_sc_ref, h_fc_ref, p_sc_ref, p_fc_ref):
  step = pl.program_id(0)
  batch = batch_ref[0, 0, :]
  onehot = (jax.lax.broadcasted_iota(jnp.int32, (G, _R), 0)
            == batch[None, :]).astype(jnp.float32)

  for b, (x_ref, wrel_ref, wroot_ref, brel_ref, h_ref, p_ref) in enumerate([
      (x_sc_ref, wrel_sc_ref, wroot_sc_ref, brel_sc_ref, h_sc_ref, p_sc_ref),
      (x_fc_ref, wrel_fc_ref, wroot_fc_ref, brel_fc_ref, h_fc_ref, p_fc_ref),
  ]):
    cnt = jnp.maximum(cnt_ref[b, :, 0], 1.0)[:, None]
    agg = sum_ref[b] / cnt
    x = x_ref[...]
    h = jnp.dot(agg, wrel_ref[...], preferred_element_type=jnp.float32)
    h = h + jnp.dot(x, wroot_ref[...], preferred_element_type=jnp.float32)
    h = jnp.maximum(h + brel_ref[...][None, :], 0.0)
    h_ref[...] = h
    part = jnp.dot(onehot, h, preferred_element_type=jnp.float32)

    @pl.when(step == 0)
    def _():
      p_ref[...] = part

    @pl.when(step != 0)
    def _():
      p_ref[...] += part


def _tc_layer(batch3, x_sc, x_fc, sums, cnts,
              wrel_sc, wroot_sc, brel_sc, wrel_fc, wroot_fc, brel_fc):
  full = lambda shape: pl.BlockSpec(shape, lambda i: tuple(0 for _ in shape))
  return pl.pallas_call(
      _tc_layer_body,
      grid=(_GRID,),
      in_specs=[
          pl.BlockSpec((1, 1, _R), lambda i: (i, 0, 0)),
          pl.BlockSpec((_R, F), lambda i: (i, 0)),
          pl.BlockSpec((_R, F), lambda i: (i, 0)),
          pl.BlockSpec((_NC, _R, F), lambda i: (0, i, 0)),
          pl.BlockSpec((_NC, _R, 1), lambda i: (0, i, 0)),
          full((F, F)), full((F, F)), full((F,)),
          full((F, F)), full((F, F)), full((F,)),
      ],
      out_specs=[
          pl.BlockSpec((_R, F), lambda i: (i, 0)),
          pl.BlockSpec((_R, F), lambda i: (i, 0)),
          pl.BlockSpec((G, F), lambda i: (0, 0)),
          pl.BlockSpec((G, F), lambda i: (0, 0)),
      ],
      out_shape=[
          jax.ShapeDtypeStruct((N, F), jnp.float32),
          jax.ShapeDtypeStruct((N, F), jnp.float32),
          jax.ShapeDtypeStruct((G, F), jnp.float32),
          jax.ShapeDtypeStruct((G, F), jnp.float32),
      ],
  )(batch3, x_sc, x_fc, sums, cnts,
    wrel_sc, wroot_sc, brel_sc, wrel_fc, wroot_fc, brel_fc)


def _tc_layer2_body(batch_ref, x_sc_ref, x_fc_ref, sum_ref, cnt_ref,
                    wrel_sc_ref, wroot_sc_ref, brel_sc_ref,
                    wrel_fc_ref, wroot_fc_ref, brel_fc_ref,
                    p1s_ref, p1f_ref,
                    w1_ref, b1_ref, w2_ref, b2_ref, w3_ref, b3_ref,
                    out_ref, p_sc_ref, p_fc_ref):
  step = pl.program_id(0)
  batch = batch_ref[0, 0, :]
  onehot = (jax.lax.broadcasted_iota(jnp.int32, (G, _R), 0)
            == batch[None, :]).astype(jnp.float32)

  for b, (x_ref, wrel_ref, wroot_ref, brel_ref, p_ref) in enumerate([
      (x_sc_ref, wrel_sc_ref, wroot_sc_ref, brel_sc_ref, p_sc_ref),
      (x_fc_ref, wrel_fc_ref, wroot_fc_ref, brel_fc_ref, p_fc_ref),
  ]):
    cnt = jnp.maximum(cnt_ref[b, :, 0], 1.0)[:, None]
    agg = sum_ref[b] / cnt
    x = x_ref[...]
    h = jnp.dot(agg, wrel_ref[...], preferred_element_type=jnp.float32)
    h = h + jnp.dot(x, wroot_ref[...], preferred_element_type=jnp.float32)
    h = jnp.maximum(h + brel_ref[...][None, :], 0.0)
    part = jnp.dot(onehot, h, preferred_element_type=jnp.float32)

    @pl.when(step == 0)
    def _():
      p_ref[...] = part

    @pl.when(step != 0)
    def _():
      p_ref[...] += part

  @pl.when(step == _GRID - 1)
  def _():
    xcat = jnp.concatenate(
        [p1s_ref[...], p_sc_ref[...], p1f_ref[...], p_fc_ref[...]], axis=1)
    x = jnp.maximum(
        jnp.dot(xcat, w1_ref[...], preferred_element_type=jnp.float32)
        + b1_ref[...][None, :], 0.0)
    x = jnp.maximum(
        jnp.dot(x, w2_ref[...], preferred_element_type=jnp.float32)
        + b2_ref[...][None, :], 0.0)
    x = (jnp.dot(x, w3_ref[...], preferred_element_type=jnp.float32)
         + b3_ref[...][None, :])
    m = jnp.max(x, axis=-1, keepdims=True)
    lse = m + jnp.log(jnp.sum(jnp.exp(x - m), axis=-1, keepdims=True))
    out_ref[...] = x - lse


def _tc_layer2(batch3, x_sc, x_fc, sums, cnts,
               wrel_sc, wroot_sc, brel_sc, wrel_fc, wroot_fc, brel_fc,
               p1s, p1f, w1, b1, w2, b2, w3, b3):
  full = lambda shape: pl.BlockSpec(shape, lambda i: tuple(0 for _ in shape))
  out, _, _ = pl.pallas_call(
      _tc_layer2_body,
      grid=(_GRID,),
      in_specs=[
          pl.BlockSpec((1, 1, _R), lambda i: (i, 0, 0)),
          pl.BlockSpec((_R, F), lambda i: (i, 0)),
          pl.BlockSpec((_R, F), lambda i: (i, 0)),
          pl.BlockSpec((_NC, _R, F), lambda i: (0, i, 0)),
          pl.BlockSpec((_NC, _R, 1), lambda i: (0, i, 0)),
          full((F, F)), full((F, F)), full((F,)),
          full((F, F)), full((F, F)), full((F,)),
          full((G, F)), full((G, F)),
          full((2 * 2 * F, F)), full((F,)),
          full((F, F // 2)), full((F // 2,)),
          full((F // 2, 2)), full((2,)),
      ],
      out_specs=[
          pl.BlockSpec((G, 2), lambda i: (0, 0)),
          pl.BlockSpec((G, F), lambda i: (0, 0)),
          pl.BlockSpec((G, F), lambda i: (0, 0)),
      ],
      out_shape=[
          jax.ShapeDtypeStruct((G, 2), jnp.float32),
          jax.ShapeDtypeStruct((G, F), jnp.float32),
          jax.ShapeDtypeStruct((G, F), jnp.float32),
      ],
  )(batch3, x_sc, x_fc, sums, cnts,
    wrel_sc, wroot_sc, brel_sc, wrel_fc, wroot_fc, brel_fc,
    p1s, p1f, w1, b1, w2, b2, w3, b3)
  return out


def _tc_head_body(p1s_ref, p2s_ref, p1f_ref, p2f_ref,
                  w1_ref, b1_ref, w2_ref, b2_ref, w3_ref, b3_ref, out_ref):
  xcat = jnp.concatenate(
      [p1s_ref[...], p2s_ref[...], p1f_ref[...], p2f_ref[...]], axis=1)
  x = jnp.maximum(
      jnp.dot(xcat, w1_ref[...], preferred_element_type=jnp.float32)
      + b1_ref[...][None, :], 0.0)
  x = jnp.maximum(
      jnp.dot(x, w2_ref[...], preferred_element_type=jnp.float32)
      + b2_ref[...][None, :], 0.0)
  x = (jnp.dot(x, w3_ref[...], preferred_element_type=jnp.float32)
       + b3_ref[...][None, :])
  m = jnp.max(x, axis=-1, keepdims=True)
  lse = m + jnp.log(jnp.sum(jnp.exp(x - m), axis=-1, keepdims=True))
  out_ref[...] = x - lse


def _tc_head(p1s, p2s, p1f, p2f, w1, b1, w2, b2, w3, b3):
  return pl.pallas_call(
      _tc_head_body,
      out_shape=jax.ShapeDtypeStruct((G, 2), jnp.float32),
  )(p1s, p2s, p1f, p2f, w1, b1, w2, b2, w3, b3)


@jax.jit
def kernel(sc_x, fc_x, sc_edge_index, fc_edge_index, batch,
           sc1_Wrel, sc1_brel, sc1_Wroot, sc2_Wrel, sc2_brel, sc2_Wroot,
           fc1_Wrel, fc1_brel, fc1_Wroot, fc2_Wrel, fc2_brel, fc2_Wroot,
           W1, b1, W2, b2, W3, b3):
  src_sc, dst_sc = sc_edge_index[0], sc_edge_index[1]
  src_fc, dst_fc = fc_edge_index[0], fc_edge_index[1]
  batch3 = batch.reshape(_GRID, 1, _R)

  sums1, cnt_flat = _sc_segsum(sc_x, fc_x, src_sc, dst_sc, src_fc, dst_fc)
  cnts = cnt_flat.reshape(_NC, N, 1)
  h1_sc, h1_fc, p1_sc, p1_fc = _tc_layer(
      batch3, sc_x, fc_x, sums1, cnts,
      sc1_Wrel, sc1_Wroot, sc1_brel, fc1_Wrel, fc1_Wroot, fc1_brel)

  sums2, _ = _sc_segsum(h1_sc, h1_fc, src_sc, dst_sc, src_fc, dst_fc,
                        with_cnt=False)
  return _tc_layer2(
      batch3, h1_sc, h1_fc, sums2, cnts,
      sc2_Wrel, sc2_Wroot, sc2_brel, fc2_Wrel, fc2_Wroot, fc2_brel,
      p1_sc, p1_fc, W1, b1, W2, b2, W3, b3)


# speedup vs baseline: 8.8262x; 1.0014x over previous
import functools
import jax
import jax.numpy as jnp
from jax import lax
from jax.experimental import pallas as pl
from jax.experimental.pallas import tpu as pltpu
from jax.experimental.pallas import tpu_sc as plsc

N = 10000
E = 320000
F = 128
G = 64

_NC = 2
_NS = 16
_K = 128
_EP = E // _NS
_NCHUNK = 156
_KT = _EP - _NCHUNK * _K
_TOFF = _NCHUNK * _K
_RT = 624
_CZ = 104
_NCOPY = _RT // _CZ
_TAIL = N - _NS * _RT


def _sc_segsum_body(with_cnt,
                    x_sc, x_fc, src_sc, dst_sc, src_fc, dst_fc,
                    out_sum, out_cnt,
                    idx0, idx1, dst0, dst1, rows0, rows1,
                    idxT, dstT, rowsT, zc1_v, ones1_v,
                    acc_sh, cnt_sh,
                    gsem0, gsem1, ssem0, ssem1, isem0, isem1, csem0, csem1):
  c = lax.axis_index("c")
  s = lax.axis_index("s")

  def _init_rows(i, _):
    def _init_lane(j, _):
      rows0[i, pl.ds(j * 16, 16)] = jnp.zeros((16,), jnp.float32)
      return 0
    lax.fori_loop(0, F // 16, _init_lane, 0)
    return 0
  lax.fori_loop(0, _CZ, _init_rows, 0)

  def _init_1d(i, _):
    zc1_v[pl.ds(i * 16, 16)] = jnp.zeros((16,), jnp.float32)
    return 0
  lax.fori_loop(0, _RT // 16, _init_1d, 0)

  if with_cnt:
    def _init_ones(i, _):
      ones1_v[pl.ds(i * 16, 16)] = jnp.ones((16,), jnp.float32)
      return 0
    lax.fori_loop(0, _K // 16, _init_ones, 0)

  for j in range(_NCOPY):
    r0 = s * _RT + j * _CZ
    pltpu.sync_copy(rows0.at[pl.ds(0, _CZ)], acc_sh.at[pl.ds(r0, _CZ)])
  if with_cnt:
    pltpu.sync_copy(zc1_v, cnt_sh.at[pl.ds(s * _RT, _RT)])

  @pl.when(s == 0)
  def _():
    pltpu.sync_copy(rows0.at[pl.ds(0, _TAIL)],
                    acc_sh.at[pl.ds(_NS * _RT, _TAIL)])
    if with_cnt:
      pltpu.sync_copy(zc1_v.at[pl.ds(0, _TAIL)],
                      cnt_sh.at[pl.ds(_NS * _RT, _TAIL)])

  plsc.subcore_barrier()

  def _process(x_ref, src_ref, dst_ref):
    bufs = ((idx0, dst0, rows0, gsem0, ssem0, isem0, csem0),
            (idx1, dst1, rows1, gsem1, ssem1, isem1, csem1))

    def idx_start(chunk, b):
      iv, dv, _, _, _, isem, _ = bufs[b]
      off = s * _EP + chunk * _K
      pltpu.async_copy(src_ref.at[pl.ds(off, _K)], iv, isem)
      pltpu.async_copy(dst_ref.at[pl.ds(off, _K)], dv, isem)

    def idx_wait(b):
      iv, dv, _, _, _, isem, _ = bufs[b]
      pltpu.make_async_copy(src_ref.at[pl.ds(0, _K)], iv, isem).wait()
      pltpu.make_async_copy(dst_ref.at[pl.ds(0, _K)], dv, isem).wait()

    def g_start(b):
      iv, _, rv, gsem, _, _, _ = bufs[b]
      pltpu.async_copy(x_ref.at[iv], rv, gsem)

    def g_wait(b):
      iv, _, rv, gsem, _, _, _ = bufs[b]
      pltpu.make_async_copy(x_ref.at[iv], rv, gsem).wait()

    def s_start(b):
      _, dv, rv, _, ssem, _, csem = bufs[b]
      pltpu.async_copy(rv, acc_sh.at[dv], ssem, add=True)
      if with_cnt:
        pltpu.async_copy(ones1_v, cnt_sh.at[dv], csem, add=True)

    def s_wait(b):
      _, dv, rv, _, ssem, _, csem = bufs[b]
      pltpu.make_async_copy(rv, acc_sh.at[dv], ssem).wait()
      if with_cnt:
        pltpu.make_async_copy(ones1_v, cnt_sh.at[dv], csem).wait()

    idx_start(0, 0)
    idx_wait(0)
    idx_start(1, 1)
    g_start(0)

    def body(j, _):
      g_wait(0)
      s_start(0)
      idx_wait(1)
      g_start(1)
      s_wait(0)

      @pl.when(2 * j + 2 < _NCHUNK)
      def _():
        idx_start(2 * j + 2, 0)

      g_wait(1)
      s_start(1)

      @pl.when(2 * j + 2 < _NCHUNK)
      def _():
        idx_wait(0)
        g_start(0)

      s_wait(1)

      @pl.when(2 * j + 3 < _NCHUNK)
      def _():
        idx_start(2 * j + 3, 1)

      return 0
    lax.fori_loop(0, _NCHUNK // 2, body, 0)

    offT = s * _EP + _TOFF
    pltpu.sync_copy(src_ref.at[pl.ds(offT, _KT)], idxT)
    pltpu.sync_copy(dst_ref.at[pl.ds(offT, _KT)], dstT)
    pltpu.async_copy(x_ref.at[idxT], rowsT, gsem0).wait()
    pltpu.sync_copy(rowsT, acc_sh.at[dstT], add=True)
    if with_cnt:
      pltpu.sync_copy(ones1_v.at[pl.ds(0, _KT)], cnt_sh.at[dstT], add=True)

  @pl.when(c == 0)
  def _():
    _process(x_sc, src_sc, dst_sc)

  @pl.when(c == 1)
  def _():
    _process(x_fc, src_fc, dst_fc)

  plsc.subcore_barrier()

  for j in range(_NCOPY):
    r0 = s * _RT + j * _CZ
    pltpu.sync_copy(acc_sh.at[pl.ds(r0, _CZ)], rows0.at[pl.ds(0, _CZ)])
    pltpu.sync_copy(rows0.at[pl.ds(0, _CZ)], out_sum.at[c, pl.ds(r0, _CZ)])
  if with_cnt:
    pltpu.sync_copy(cnt_sh.at[pl.ds(s * _RT, _RT)], zc1_v)
    pltpu.sync_copy(zc1_v, out_cnt.at[pl.ds(c * N + s * _RT, _RT)])

  @pl.when(s == 0)
  def _():
    r0 = _NS * _RT
    pltpu.sync_copy(acc_sh.at[pl.ds(r0, _TAIL)], rows0.at[pl.ds(0, _TAIL)])
    pltpu.sync_copy(rows0.at[pl.ds(0, _TAIL)], out_sum.at[c, pl.ds(r0, _TAIL)])
    if with_cnt:
      pltpu.sync_copy(cnt_sh.at[pl.ds(r0, _TAIL)], zc1_v.at[pl.ds(0, _TAIL)])
      pltpu.sync_copy(zc1_v.at[pl.ds(0, _TAIL)],
                      out_cnt.at[pl.ds(c * N + r0, _TAIL)])


def _sc_segsum(x_sc, x_fc, src_sc, dst_sc, src_fc, dst_fc, with_cnt=True):
  mesh = plsc.VectorSubcoreMesh(core_axis_name="c", subcore_axis_name="s",
                                num_cores=_NC, num_subcores=_NS)
  f = pl.kernel(
      functools.partial(_sc_segsum_body, with_cnt),
      mesh=mesh,
      out_type=[
          jax.ShapeDtypeStruct((_NC, N, F), jnp.float32),
          jax.ShapeDtypeStruct((_NC * N,), jnp.float32),
      ],
      scratch_types=[
          pltpu.VMEM((_K,), jnp.int32),
          pltpu.VMEM((_K,), jnp.int32),
          pltpu.VMEM((_K,), jnp.int32),
          pltpu.VMEM((_K,), jnp.int32),
          pltpu.VMEM((_K, F), jnp.float32),
          pltpu.VMEM((_K, F), jnp.float32),
          pltpu.VMEM((_KT,), jnp.int32),
          pltpu.VMEM((_KT,), jnp.int32),
          pltpu.VMEM((_KT, F), jnp.float32),
          pltpu.VMEM((_RT,), jnp.float32),
          pltpu.VMEM((_K,), jnp.float32),
          pltpu.VMEM_SHARED((N, F), jnp.float32),
          pltpu.VMEM_SHARED((N,), jnp.float32),
          pltpu.SemaphoreType.DMA,
          pltpu.SemaphoreType.DMA,
          pltpu.SemaphoreType.DMA,
          pltpu.SemaphoreType.DMA,
          pltpu.SemaphoreType.DMA,
          pltpu.SemaphoreType.DMA,
          pltpu.SemaphoreType.DMA,
          pltpu.SemaphoreType.DMA,
      ],
  )
  return f(x_sc, x_fc, src_sc, dst_sc, src_fc, dst_fc)


_R = 5000
_GRID = N // _R


def _tc_layer_body(batch_ref, x_sc_ref, x_fc_ref, sum_ref, cnt_ref,
                   wrel_sc_ref, wroot_sc_ref, brel_sc_ref,
                   wrel_fc_ref, wroot_fc_ref, brel_fc_ref,
                   h_sc_ref, h_fc_ref, p_sc_ref, p_fc_ref):
  step = pl.program_id(0)
  batch = batch_ref[0, 0, :]
  onehot = (jax.lax.broadcasted_iota(jnp.int32, (G, _R), 0)
            == batch[None, :]).astype(jnp.float32)

  for b, (x_ref, wrel_ref, wroot_ref, brel_ref, h_ref, p_ref) in enumerate([
      (x_sc_ref, wrel_sc_ref, wroot_sc_ref, brel_sc_ref, h_sc_ref, p_sc_ref),
      (x_fc_ref, wrel_fc_ref, wroot_fc_ref, brel_fc_ref, h_fc_ref, p_fc_ref),
  ]):
    cnt = jnp.maximum(cnt_ref[b, :, 0], 1.0)[:, None]
    agg = sum_ref[b] / cnt
    x = x_ref[...]
    h = jnp.dot(agg, wrel_ref[...], preferred_element_type=jnp.float32)
    h = h + jnp.dot(x, wroot_ref[...], preferred_element_type=jnp.float32)
    h = jnp.maximum(h + brel_ref[...][None, :], 0.0)
    h_ref[...] = h
    part = jnp.dot(onehot, h, preferred_element_type=jnp.float32)

    @pl.when(step == 0)
    def _():
      p_ref[...] = part

    @pl.when(step != 0)
    def _():
      p_ref[...] += part


def _tc_layer(batch3, x_sc, x_fc, sums, cnts,
              wrel_sc, wroot_sc, brel_sc, wrel_fc, wroot_fc, brel_fc):
  full = lambda shape: pl.BlockSpec(shape, lambda i: tuple(0 for _ in shape))
  return pl.pallas_call(
      _tc_layer_body,
      grid=(_GRID,),
      in_specs=[
          pl.BlockSpec((1, 1, _R), lambda i: (i, 0, 0)),
          pl.BlockSpec((_R, F), lambda i: (i, 0)),
          pl.BlockSpec((_R, F), lambda i: (i, 0)),
          pl.BlockSpec((_NC, _R, F), lambda i: (0, i, 0)),
          pl.BlockSpec((_NC, _R, 1), lambda i: (0, i, 0)),
          full((F, F)), full((F, F)), full((F,)),
          full((F, F)), full((F, F)), full((F,)),
      ],
      out_specs=[
          pl.BlockSpec((_R, F), lambda i: (i, 0)),
          pl.BlockSpec((_R, F), lambda i: (i, 0)),
          pl.BlockSpec((G, F), lambda i: (0, 0)),
          pl.BlockSpec((G, F), lambda i: (0, 0)),
      ],
      out_shape=[
          jax.ShapeDtypeStruct((N, F), jnp.float32),
          jax.ShapeDtypeStruct((N, F), jnp.float32),
          jax.ShapeDtypeStruct((G, F), jnp.float32),
          jax.ShapeDtypeStruct((G, F), jnp.float32),
      ],
  )(batch3, x_sc, x_fc, sums, cnts,
    wrel_sc, wroot_sc, brel_sc, wrel_fc, wroot_fc, brel_fc)


def _tc_layer2_body(batch_ref, x_sc_ref, x_fc_ref, sum_ref, cnt_ref,
                    wrel_sc_ref, wroot_sc_ref, brel_sc_ref,
                    wrel_fc_ref, wroot_fc_ref, brel_fc_ref,
                    p1s_ref, p1f_ref,
                    w1_ref, b1_ref, w2_ref, b2_ref, w3_ref, b3_ref,
                    out_ref, p_sc_ref, p_fc_ref):
  step = pl.program_id(0)
  batch = batch_ref[0, 0, :]
  onehot = (jax.lax.broadcasted_iota(jnp.int32, (G, _R), 0)
            == batch[None, :]).astype(jnp.float32)

  for b, (x_ref, wrel_ref, wroot_ref, brel_ref, p_ref) in enumerate([
      (x_sc_ref, wrel_sc_ref, wroot_sc_ref, brel_sc_ref, p_sc_ref),
      (x_fc_ref, wrel_fc_ref, wroot_fc_ref, brel_fc_ref, p_fc_ref),
  ]):
    cnt = jnp.maximum(cnt_ref[b, :, 0], 1.0)[:, None]
    agg = sum_ref[b] / cnt
    x = x_ref[...]
    h = jnp.dot(agg, wrel_ref[...], preferred_element_type=jnp.float32)
    h = h + jnp.dot(x, wroot_ref[...], preferred_element_type=jnp.float32)
    h = jnp.maximum(h + brel_ref[...][None, :], 0.0)
    part = jnp.dot(onehot, h, preferred_element_type=jnp.float32)

    @pl.when(step == 0)
    def _():
      p_ref[...] = part

    @pl.when(step != 0)
    def _():
      p_ref[...] += part

  @pl.when(step == _GRID - 1)
  def _():
    xcat = jnp.concatenate(
        [p1s_ref[...], p_sc_ref[...], p1f_ref[...], p_fc_ref[...]], axis=1)
    x = jnp.maximum(
        jnp.dot(xcat, w1_ref[...], preferred_element_type=jnp.float32)
        + b1_ref[...][None, :], 0.0)
    x = jnp.maximum(
        jnp.dot(x, w2_ref[...], preferred_element_type=jnp.float32)
        + b2_ref[...][None, :], 0.0)
    x = (jnp.dot(x, w3_ref[...], preferred_element_type=jnp.float32)
         + b3_ref[...][None, :])
    m = jnp.max(x, axis=-1, keepdims=True)
    lse = m + jnp.log(jnp.sum(jnp.exp(x - m), axis=-1, keepdims=True))
    out_ref[...] = x - lse


def _tc_layer2(batch3, x_sc, x_fc, sums, cnts,
               wrel_sc, wroot_sc, brel_sc, wrel_fc, wroot_fc, brel_fc,
               p1s, p1f, w1, b1, w2, b2, w3, b3):
  full = lambda shape: pl.BlockSpec(shape, lambda i: tuple(0 for _ in shape))
  out, _, _ = pl.pallas_call(
      _tc_layer2_body,
      grid=(_GRID,),
      in_specs=[
          pl.BlockSpec((1, 1, _R), lambda i: (i, 0, 0)),
          pl.BlockSpec((_R, F), lambda i: (i, 0)),
          pl.BlockSpec((_R, F), lambda i: (i, 0)),
          pl.BlockSpec((_NC, _R, F), lambda i: (0, i, 0)),
          pl.BlockSpec((_NC, _R, 1), lambda i: (0, i, 0)),
          full((F, F)), full((F, F)), full((F,)),
          full((F, F)), full((F, F)), full((F,)),
          full((G, F)), full((G, F)),
          full((2 * 2 * F, F)), full((F,)),
          full((F, F // 2)), full((F // 2,)),
          full((F // 2, 2)), full((2,)),
      ],
      out_specs=[
          pl.BlockSpec((G, 2), lambda i: (0, 0)),
          pl.BlockSpec((G, F), lambda i: (0, 0)),
          pl.BlockSpec((G, F), lambda i: (0, 0)),
      ],
      out_shape=[
          jax.ShapeDtypeStruct((G, 2), jnp.float32),
          jax.ShapeDtypeStruct((G, F), jnp.float32),
          jax.ShapeDtypeStruct((G, F), jnp.float32),
      ],
  )(batch3, x_sc, x_fc, sums, cnts,
    wrel_sc, wroot_sc, brel_sc, wrel_fc, wroot_fc, brel_fc,
    p1s, p1f, w1, b1, w2, b2, w3, b3)
  return out


def _tc_head_body(p1s_ref, p2s_ref, p1f_ref, p2f_ref,
                  w1_ref, b1_ref, w2_ref, b2_ref, w3_ref, b3_ref, out_ref):
  xcat = jnp.concatenate(
      [p1s_ref[...], p2s_ref[...], p1f_ref[...], p2f_ref[...]], axis=1)
  x = jnp.maximum(
      jnp.dot(xcat, w1_ref[...], preferred_element_type=jnp.float32)
      + b1_ref[...][None, :], 0.0)
  x = jnp.maximum(
      jnp.dot(x, w2_ref[...], preferred_element_type=jnp.float32)
      + b2_ref[...][None, :], 0.0)
  x = (jnp.dot(x, w3_ref[...], preferred_element_type=jnp.float32)
       + b3_ref[...][None, :])
  m = jnp.max(x, axis=-1, keepdims=True)
  lse = m + jnp.log(jnp.sum(jnp.exp(x - m), axis=-1, keepdims=True))
  out_ref[...] = x - lse


def _tc_head(p1s, p2s, p1f, p2f, w1, b1, w2, b2, w3, b3):
  return pl.pallas_call(
      _tc_head_body,
      out_shape=jax.ShapeDtypeStruct((G, 2), jnp.float32),
  )(p1s, p2s, p1f, p2f, w1, b1, w2, b2, w3, b3)


@jax.jit
def kernel(sc_x, fc_x, sc_edge_index, fc_edge_index, batch,
           sc1_Wrel, sc1_brel, sc1_Wroot, sc2_Wrel, sc2_brel, sc2_Wroot,
           fc1_Wrel, fc1_brel, fc1_Wroot, fc2_Wrel, fc2_brel, fc2_Wroot,
           W1, b1, W2, b2, W3, b3):
  src_sc, dst_sc = sc_edge_index[0], sc_edge_index[1]
  src_fc, dst_fc = fc_edge_index[0], fc_edge_index[1]
  batch3 = batch.reshape(_GRID, 1, _R)

  sums1, cnt_flat = _sc_segsum(sc_x, fc_x, src_sc, dst_sc, src_fc, dst_fc)
  cnts = cnt_flat.reshape(_NC, N, 1)
  h1_sc, h1_fc, p1_sc, p1_fc = _tc_layer(
      batch3, sc_x, fc_x, sums1, cnts,
      sc1_Wrel, sc1_Wroot, sc1_brel, fc1_Wrel, fc1_Wroot, fc1_brel)

  sums2, _ = _sc_segsum(h1_sc, h1_fc, src_sc, dst_sc, src_fc, dst_fc,
                        with_cnt=False)
  return _tc_layer2(
      batch3, h1_sc, h1_fc, sums2, cnts,
      sc2_Wrel, sc2_Wroot, sc2_brel, fc2_Wrel, fc2_Wroot, fc2_brel,
      p1_sc, p1_fc, W1, b1, W2, b2, W3, b3)


# speedup vs baseline: 8.8405x; 1.0016x over previous
import functools
import jax
import jax.numpy as jnp
from jax import lax
from jax.experimental import pallas as pl
from jax.experimental.pallas import tpu as pltpu
from jax.experimental.pallas import tpu_sc as plsc

N = 10000
E = 320000
F = 128
G = 64

_NC = 2
_NS = 16
_K = 128
_EP = E // _NS
_NCHUNK = 156
_KT = _EP - _NCHUNK * _K
_TOFF = _NCHUNK * _K
_RT = 624
_CZ = 104
_NCOPY = _RT // _CZ
_TAIL = N - _NS * _RT


def _sc_segsum_body(with_cnt,
                    x_sc, x_fc, src_sc, dst_sc, src_fc, dst_fc,
                    out_sum, out_cnt,
                    idx0, idx1, dst0, dst1, rows0, rows1,
                    idxT, dstT, rowsT, zc1_v, ones1_v,
                    acc_sh, cnt_sh,
                    gsem0, gsem1, ssem0, ssem1, isem0, isem1, csem0, csem1):
  c = lax.axis_index("c")
  s = lax.axis_index("s")

  def _init_rows(i, _):
    def _init_lane(j, _):
      rows0[i, pl.ds(j * 16, 16)] = jnp.zeros((16,), jnp.float32)
      return 0
    lax.fori_loop(0, F // 16, _init_lane, 0)
    return 0
  lax.fori_loop(0, _CZ, _init_rows, 0)

  def _init_1d(i, _):
    zc1_v[pl.ds(i * 16, 16)] = jnp.zeros((16,), jnp.float32)
    return 0
  lax.fori_loop(0, _RT // 16, _init_1d, 0)

  if with_cnt:
    def _init_ones(i, _):
      ones1_v[pl.ds(i * 16, 16)] = jnp.ones((16,), jnp.float32)
      return 0
    lax.fori_loop(0, _K // 16, _init_ones, 0)

  for j in range(_NCOPY):
    r0 = s * _RT + j * _CZ
    pltpu.sync_copy(rows0.at[pl.ds(0, _CZ)], acc_sh.at[pl.ds(r0, _CZ)])
  if with_cnt:
    pltpu.sync_copy(zc1_v, cnt_sh.at[pl.ds(s * _RT, _RT)])

  @pl.when(s == 0)
  def _():
    pltpu.sync_copy(rows0.at[pl.ds(0, _TAIL)],
                    acc_sh.at[pl.ds(_NS * _RT, _TAIL)])
    if with_cnt:
      pltpu.sync_copy(zc1_v.at[pl.ds(0, _TAIL)],
                      cnt_sh.at[pl.ds(_NS * _RT, _TAIL)])

  plsc.subcore_barrier()

  def _process(x_ref, src_ref, dst_ref):
    bufs = ((idx0, dst0, rows0, gsem0, ssem0, isem0, csem0),
            (idx1, dst1, rows1, gsem1, ssem1, isem1, csem1))

    def idx_start(chunk, b):
      iv, dv, _, _, _, isem, _ = bufs[b]
      off = s * _EP + chunk * _K
      pltpu.async_copy(src_ref.at[pl.ds(off, _K)], iv, isem)
      pltpu.async_copy(dst_ref.at[pl.ds(off, _K)], dv, isem)

    def idx_wait(b):
      iv, dv, _, _, _, isem, _ = bufs[b]
      pltpu.make_async_copy(src_ref.at[pl.ds(0, _K)], iv, isem).wait()
      pltpu.make_async_copy(dst_ref.at[pl.ds(0, _K)], dv, isem).wait()

    def g_start(b):
      iv, _, rv, gsem, _, _, _ = bufs[b]
      pltpu.async_copy(x_ref.at[iv], rv, gsem)

    def g_wait(b):
      iv, _, rv, gsem, _, _, _ = bufs[b]
      pltpu.make_async_copy(x_ref.at[iv], rv, gsem).wait()

    def s_start(b):
      _, dv, rv, _, ssem, _, csem = bufs[b]
      pltpu.async_copy(rv, acc_sh.at[dv], ssem, add=True)
      if with_cnt:
        pltpu.async_copy(ones1_v, cnt_sh.at[dv], csem, add=True)

    def s_wait(b):
      _, dv, rv, _, ssem, _, csem = bufs[b]
      pltpu.make_async_copy(rv, acc_sh.at[dv], ssem).wait()
      if with_cnt:
        pltpu.make_async_copy(ones1_v, cnt_sh.at[dv], csem).wait()

    idx_start(0, 0)
    idx_wait(0)
    idx_start(1, 1)
    g_start(0)

    def body(j, _):
      g_wait(0)
      s_start(0)
      idx_wait(1)
      g_start(1)
      s_wait(0)

      @pl.when(2 * j + 2 < _NCHUNK)
      def _():
        idx_start(2 * j + 2, 0)

      g_wait(1)
      s_start(1)

      @pl.when(2 * j + 2 < _NCHUNK)
      def _():
        idx_wait(0)
        g_start(0)

      s_wait(1)

      @pl.when(2 * j + 3 < _NCHUNK)
      def _():
        idx_start(2 * j + 3, 1)

      return 0
    lax.fori_loop(0, _NCHUNK // 2, body, 0)

    offT = s * _EP + _TOFF
    pltpu.sync_copy(src_ref.at[pl.ds(offT, _KT)], idxT)
    pltpu.sync_copy(dst_ref.at[pl.ds(offT, _KT)], dstT)
    pltpu.async_copy(x_ref.at[idxT], rowsT, gsem0).wait()
    pltpu.sync_copy(rowsT, acc_sh.at[dstT], add=True)
    if with_cnt:
      pltpu.sync_copy(ones1_v.at[pl.ds(0, _KT)], cnt_sh.at[dstT], add=True)

  @pl.when(c == 0)
  def _():
    _process(x_sc, src_sc, dst_sc)

  @pl.when(c == 1)
  def _():
    _process(x_fc, src_fc, dst_fc)

  plsc.subcore_barrier()

  for j in range(_NCOPY):
    r0 = s * _RT + j * _CZ
    pltpu.sync_copy(acc_sh.at[pl.ds(r0, _CZ)], rows0.at[pl.ds(0, _CZ)])
    pltpu.sync_copy(rows0.at[pl.ds(0, _CZ)], out_sum.at[c, pl.ds(r0, _CZ)])
  if with_cnt:
    pltpu.sync_copy(cnt_sh.at[pl.ds(s * _RT, _RT)], zc1_v)
    pltpu.sync_copy(zc1_v, out_cnt.at[pl.ds(c * N + s * _RT, _RT)])

  @pl.when(s == 0)
  def _():
    r0 = _NS * _RT
    pltpu.sync_copy(acc_sh.at[pl.ds(r0, _TAIL)], rows0.at[pl.ds(0, _TAIL)])
    pltpu.sync_copy(rows0.at[pl.ds(0, _TAIL)], out_sum.at[c, pl.ds(r0, _TAIL)])
    if with_cnt:
      pltpu.sync_copy(cnt_sh.at[pl.ds(r0, _TAIL)], zc1_v.at[pl.ds(0, _TAIL)])
      pltpu.sync_copy(zc1_v.at[pl.ds(0, _TAIL)],
                      out_cnt.at[pl.ds(c * N + r0, _TAIL)])


def _sc_segsum(x_sc, x_fc, src_sc, dst_sc, src_fc, dst_fc, with_cnt=True):
  mesh = plsc.VectorSubcoreMesh(core_axis_name="c", subcore_axis_name="s",
                                num_cores=_NC, num_subcores=_NS)
  f = pl.kernel(
      functools.partial(_sc_segsum_body, with_cnt),
      mesh=mesh,
      out_type=[
          jax.ShapeDtypeStruct((_NC, N, F), jnp.float32),
          jax.ShapeDtypeStruct((_NC * N,), jnp.float32),
      ],
      scratch_types=[
          pltpu.VMEM((_K,), jnp.int32),
          pltpu.VMEM((_K,), jnp.int32),
          pltpu.VMEM((_K,), jnp.int32),
          pltpu.VMEM((_K,), jnp.int32),
          pltpu.VMEM((_K, F), jnp.float32),
          pltpu.VMEM((_K, F), jnp.float32),
          pltpu.VMEM((_KT,), jnp.int32),
          pltpu.VMEM((_KT,), jnp.int32),
          pltpu.VMEM((_KT, F), jnp.float32),
          pltpu.VMEM((_RT,), jnp.float32),
          pltpu.VMEM((_K,), jnp.float32),
          pltpu.VMEM_SHARED((N, F), jnp.float32),
          pltpu.VMEM_SHARED((N,), jnp.float32),
          pltpu.SemaphoreType.DMA,
          pltpu.SemaphoreType.DMA,
          pltpu.SemaphoreType.DMA,
          pltpu.SemaphoreType.DMA,
          pltpu.SemaphoreType.DMA,
          pltpu.SemaphoreType.DMA,
          pltpu.SemaphoreType.DMA,
          pltpu.SemaphoreType.DMA,
      ],
  )
  return f(x_sc, x_fc, src_sc, dst_sc, src_fc, dst_fc)


_R = 5000
_GRID = N // _R


def _tc_layer_body(batch_ref, x_sc_ref, x_fc_ref, sum_ref, cnt_ref,
                   wrel_sc_ref, wroot_sc_ref, brel_sc_ref,
                   wrel_fc_ref, wroot_fc_ref, brel_fc_ref,
                   h_sc_ref, h_fc_ref, p_sc_ref, p_fc_ref):
  step = pl.program_id(0)
  batch = batch_ref[0, 0, :]
  onehot = (jax.lax.broadcasted_iota(jnp.int32, (G, _R), 0)
            == batch[None, :]).astype(jnp.float32)

  for b, (x_ref, wrel_ref, wroot_ref, brel_ref, h_ref, p_ref) in enumerate([
      (x_sc_ref, wrel_sc_ref, wroot_sc_ref, brel_sc_ref, h_sc_ref, p_sc_ref),
      (x_fc_ref, wrel_fc_ref, wroot_fc_ref, brel_fc_ref, h_fc_ref, p_fc_ref),
  ]):
    cnt = jnp.maximum(cnt_ref[b, :, 0], 1.0)[:, None]
    agg = sum_ref[b] / cnt
    x = x_ref[...]
    h = jnp.dot(agg, wrel_ref[...], preferred_element_type=jnp.float32)
    h = h + jnp.dot(x, wroot_ref[...], preferred_element_type=jnp.float32)
    h = jnp.maximum(h + brel_ref[...][None, :], 0.0)
    h_ref[...] = h
    part = jnp.dot(onehot, h, preferred_element_type=jnp.float32)

    @pl.when(step == 0)
    def _():
      p_ref[...] = part

    @pl.when(step != 0)
    def _():
      p_ref[...] += part


def _tc_layer(batch3, x_sc, x_fc, sums, cnts,
              wrel_sc, wroot_sc, brel_sc, wrel_fc, wroot_fc, brel_fc):
  full = lambda shape: pl.BlockSpec(shape, lambda i: tuple(0 for _ in shape))
  return pl.pallas_call(
      _tc_layer_body,
      grid=(_GRID,),
      in_specs=[
          pl.BlockSpec((1, 1, _R), lambda i: (i, 0, 0)),
          pl.BlockSpec((_R, F), lambda i: (i, 0)),
          pl.BlockSpec((_R, F), lambda i: (i, 0)),
          pl.BlockSpec((_NC, _R, F), lambda i: (0, i, 0)),
          pl.BlockSpec((_NC, _R, 1), lambda i: (0, i, 0)),
          full((F, F)), full((F, F)), full((F,)),
          full((F, F)), full((F, F)), full((F,)),
      ],
      out_specs=[
          pl.BlockSpec((_R, F), lambda i: (i, 0)),
          pl.BlockSpec((_R, F), lambda i: (i, 0)),
          pl.BlockSpec((G, F), lambda i: (0, 0)),
          pl.BlockSpec((G, F), lambda i: (0, 0)),
      ],
      out_shape=[
          jax.ShapeDtypeStruct((N, F), jnp.float32),
          jax.ShapeDtypeStruct((N, F), jnp.float32),
          jax.ShapeDtypeStruct((G, F), jnp.float32),
          jax.ShapeDtypeStruct((G, F), jnp.float32),
      ],
  )(batch3, x_sc, x_fc, sums, cnts,
    wrel_sc, wroot_sc, brel_sc, wrel_fc, wroot_fc, brel_fc)


def _tc_layer2_body(batch_ref, x_sc_ref, x_fc_ref, sum_ref, cnt_ref,
                    wrel_sc_ref, wroot_sc_ref, brel_sc_ref,
                    wrel_fc_ref, wroot_fc_ref, brel_fc_ref,
                    p1s_ref, p1f_ref,
                    w1_ref, b1_ref, w2_ref, b2_ref, w3_ref, b3_ref,
                    out_ref, p_sc_ref, p_fc_ref):
  step = pl.program_id(0)
  batch = batch_ref[0, 0, :]
  onehot = (jax.lax.broadcasted_iota(jnp.int32, (G, _R), 0)
            == batch[None, :]).astype(jnp.float32)

  for b, (x_ref, wrel_ref, wroot_ref, brel_ref, p_ref) in enumerate([
      (x_sc_ref, wrel_sc_ref, wroot_sc_ref, brel_sc_ref, p_sc_ref),
      (x_fc_ref, wrel_fc_ref, wroot_fc_ref, brel_fc_ref, p_fc_ref),
  ]):
    cnt = jnp.maximum(cnt_ref[b, :, 0], 1.0)[:, None]
    agg = sum_ref[b] / cnt
    x = x_ref[...]
    h = jnp.dot(agg, wrel_ref[...], preferred_element_type=jnp.float32)
    h = h + jnp.dot(x, wroot_ref[...], preferred_element_type=jnp.float32)
    h = jnp.maximum(h + brel_ref[...][None, :], 0.0)
    part = jnp.dot(onehot, h, preferred_element_type=jnp.float32)

    @pl.when(step == 0)
    def _():
      p_ref[...] = part

    @pl.when(step != 0)
    def _():
      p_ref[...] += part

  @pl.when(step == _GRID - 1)
  def _():
    xcat = jnp.concatenate(
        [p1s_ref[...], p_sc_ref[...], p1f_ref[...], p_fc_ref[...]], axis=1)
    x = jnp.maximum(
        jnp.dot(xcat, w1_ref[...], preferred_element_type=jnp.float32)
        + b1_ref[...][None, :], 0.0)
    x = jnp.maximum(
        jnp.dot(x, w2_ref[...], preferred_element_type=jnp.float32)
        + b2_ref[...][None, :], 0.0)
    x = (jnp.dot(x, w3_ref[...], preferred_element_type=jnp.float32)
         + b3_ref[...][None, :])
    m = jnp.max(x, axis=-1, keepdims=True)
    lse = m + jnp.log(jnp.sum(jnp.exp(x - m), axis=-1, keepdims=True))
    out_ref[...] = x - lse


def _tc_layer2(batch3, x_sc, x_fc, sums, cnts,
               wrel_sc, wroot_sc, brel_sc, wrel_fc, wroot_fc, brel_fc,
               p1s, p1f, w1, b1, w2, b2, w3, b3):
  full = lambda shape: pl.BlockSpec(shape, lambda i: tuple(0 for _ in shape))
  out, _, _ = pl.pallas_call(
      _tc_layer2_body,
      grid=(_GRID,),
      in_specs=[
          pl.BlockSpec((1, 1, _R), lambda i: (i, 0, 0)),
          pl.BlockSpec((_R, F), lambda i: (i, 0)),
          pl.BlockSpec((_R, F), lambda i: (i, 0)),
          pl.BlockSpec((_NC, _R, F), lambda i: (0, i, 0)),
          pl.BlockSpec((_NC, _R, 1), lambda i: (0, i, 0)),
          full((F, F)), full((F, F)), full((F,)),
          full((F, F)), full((F, F)), full((F,)),
          full((G, F)), full((G, F)),
          full((2 * 2 * F, F)), full((F,)),
          full((F, F // 2)), full((F // 2,)),
          full((F // 2, 2)), full((2,)),
      ],
      out_specs=[
          pl.BlockSpec((G, 2), lambda i: (0, 0)),
          pl.BlockSpec((G, F), lambda i: (0, 0)),
          pl.BlockSpec((G, F), lambda i: (0, 0)),
      ],
      out_shape=[
          jax.ShapeDtypeStruct((G, 2), jnp.float32),
          jax.ShapeDtypeStruct((G, F), jnp.float32),
          jax.ShapeDtypeStruct((G, F), jnp.float32),
      ],
  )(batch3, x_sc, x_fc, sums, cnts,
    wrel_sc, wroot_sc, brel_sc, wrel_fc, wroot_fc, brel_fc,
    p1s, p1f, w1, b1, w2, b2, w3, b3)
  return out


@jax.jit
def kernel(sc_x, fc_x, sc_edge_index, fc_edge_index, batch,
           sc1_Wrel, sc1_brel, sc1_Wroot, sc2_Wrel, sc2_brel, sc2_Wroot,
           fc1_Wrel, fc1_brel, fc1_Wroot, fc2_Wrel, fc2_brel, fc2_Wroot,
           W1, b1, W2, b2, W3, b3):
  src_sc, dst_sc = sc_edge_index[0], sc_edge_index[1]
  src_fc, dst_fc = fc_edge_index[0], fc_edge_index[1]
  batch3 = batch.reshape(_GRID, 1, _R)

  sums1, cnt_flat = _sc_segsum(sc_x, fc_x, src_sc, dst_sc, src_fc, dst_fc)
  cnts = cnt_flat.reshape(_NC, N, 1)
  h1_sc, h1_fc, p1_sc, p1_fc = _tc_layer(
      batch3, sc_x, fc_x, sums1, cnts,
      sc1_Wrel, sc1_Wroot, sc1_brel, fc1_Wrel, fc1_Wroot, fc1_brel)

  sums2, _ = _sc_segsum(h1_sc, h1_fc, src_sc, dst_sc, src_fc, dst_fc,
                        with_cnt=False)
  return _tc_layer2(
      batch3, h1_sc, h1_fc, sums2, cnts,
      sc2_Wrel, sc2_Wroot, sc2_brel, fc2_Wrel, fc2_Wroot, fc2_brel,
      p1_sc, p1_fc, W1, b1, W2, b2, W3, b3)
